# Initial kernel scaffold; baseline (speedup 1.0000x reference)
import jax
import jax.numpy as jnp
from jax.experimental import pallas as pl


def kernel(pos, edge_index, batch, W1a, b1a, W2a, b2a, W1b, b1b, W2b, b2b, Wc, bc):
    raise NotImplementedError("write your pallas kernel here")


# speedup vs baseline: 1.0023x; 1.0023x over previous
import jax
import jax.numpy as jnp
from jax.experimental import pallas as pl

N = 100000
G = 64


def _layer(h, pos, src, dst, W1, b1, W2, b2):
    msg_in = jnp.concatenate([h[src], pos[src] - pos[dst]], axis=-1)
    m = jnp.maximum(msg_in @ W1 + b1, 0.0) @ W2 + b2
    agg = jax.ops.segment_max(m, dst, num_segments=N)
    return jnp.where(jnp.isfinite(agg), agg, 0.0)


def _final_body(g_ref, wc_ref, bc_ref, out_ref):
    out_ref[...] = g_ref[...] @ wc_ref[...] + bc_ref[...]


def kernel(pos, edge_index, batch, W1a, b1a, W2a, b2a, W1b, b1b, W2b, b2b, Wc, bc):
    src, dst = edge_index[0], edge_index[1]
    h = jnp.maximum(_layer(pos, pos, src, dst, W1a, b1a, W2a, b2a), 0.0)
    h = jnp.maximum(_layer(h, pos, src, dst, W1b, b1b, W2b, b2b), 0.0)
    g = jax.ops.segment_max(h, batch, num_segments=G)
    g = jnp.where(jnp.isfinite(g), g, 0.0)
    return pl.pallas_call(
        _final_body,
        out_shape=jax.ShapeDtypeStruct((G, Wc.shape[1]), jnp.float32),
    )(g, Wc, bc)


# speedup vs baseline: 1.3353x; 1.3323x over previous
import functools

import jax
import jax.numpy as jnp
from jax import lax
from jax.experimental import pallas as pl
from jax.experimental.pallas import tpu as pltpu
from jax.experimental.pallas import tpu_sc as plsc

NN = 100000
EE = 1600000
GG = 64
F = 32
NC, NS, LN = 2, 16, 16
NW = NC * NS
NB = 32
CSP = 3200
NPAD = NB * CSP
EPW = EE // NW
CH = 2000
NCH = EPW // CH
EPAD = EE + 8192
EPW2 = EPAD // NW
S2CB = 512
S2NF = EPW2 // S2CB
S2REM = EPW2 - S2NF * S2CB
ESZ = EPAD + 128
BR1 = 6400
BR3 = 4608

_SC_PARAMS = pltpu.CompilerParams(
    needs_layout_passes=False, use_tc_tiling_on_sc=False)


def _mesh():
    return plsc.VectorSubcoreMesh(
        core_axis_name="c", subcore_axis_name="s",
        num_cores=NC, num_subcores=NS)


def _wid():
    return lax.axis_index("s") * NC + lax.axis_index("c")


def _bkt(d):
    return ((d >> 7) * 5243) >> 17


def _lanes():
    return lax.iota(jnp.int32, LN)


def _prefix(cb, cs):
    lanes = _lanes()
    carry = jnp.int32(0)
    for k in range(NW * NB // LN):
        j = k * LN + lanes
        b = j >> 5
        wv = j & 31
        idx = wv * NB + b
        c = plsc.load_gather(cb, [idx])
        cp = (c + 7) & (-8)
        excl = plsc.cumsum(cp) - cp + carry
        plsc.store_scatter(cs, [idx], excl)
        carry = carry + jnp.sum(cp)
    return carry


def _pa_body(dst_hbm, counts_hbm, dbuf, hist, cbuf, sem):
    del sem
    w = _wid()
    lanes = _lanes()
    for b in range(NB):
        hist[pl.ds(b * LN, LN)] = jnp.zeros((LN,), jnp.int32)

    def chunk(c, car):
        pltpu.sync_copy(dst_hbm.at[pl.ds(pl.multiple_of(w * EPW + c * CH, 8), CH)], dbuf)

        def hb(v, car2):
            d = dbuf[pl.ds(v * LN, LN)]
            fi = _bkt(d) * LN + lanes
            plsc.store_scatter(hist, [fi], plsc.load_gather(hist, [fi]) + 1)
            return car2

        return lax.fori_loop(0, CH // LN, hb, car)

    lax.fori_loop(0, NCH, chunk, 0)
    for g in range(NB // LN):
        acc = jnp.zeros((LN,), jnp.int32)
        for l in range(LN):
            acc = acc + plsc.load_gather(hist, [(lanes + g * LN) * LN + l])
        cbuf[pl.ds(g * LN, LN)] = acc
    pltpu.sync_copy(cbuf, counts_hbm.at[pl.ds(pl.multiple_of(w * NB, 8), NB)])


def _run_pa(dst):
    kfn = functools.partial(
        pl.kernel, mesh=_mesh(), compiler_params=_SC_PARAMS,
        out_type=jax.ShapeDtypeStruct((NW * NB,), jnp.int32),
        scratch_types=[
            pltpu.VMEM((CH,), jnp.int32),
            pltpu.VMEM((NB * LN,), jnp.int32),
            pltpu.VMEM((NB,), jnp.int32),
            pltpu.SemaphoreType.DMA,
        ],
    )(_pa_body)
    return kfn(dst)


def _pc_body(src_hbm, dst_hbm, counts_hbm, srcp_hbm, dstp_hbm,
             dbuf, sbuf, cb, cs, fl, cs0, stash_s, stash_d,
             posbuf, vbs, vbd, idxp, valp_s, valp_d, sem):
    w = _wid()
    lanes = _lanes()
    pltpu.sync_copy(counts_hbm, cb)
    _prefix(cb, cs)
    for g in range(NB // LN):
        row = cs[pl.ds(w * NB + g * LN, LN)]
        fl[pl.ds(g * LN, LN)] = row
        cs0[pl.ds(g * LN, LN)] = row
    for r in range(16):
        for k in range(8):
            posbuf[r, pl.ds(k * LN, LN)] = jnp.full((LN,), EPAD, jnp.int32)
            vbs[r, pl.ds(k * LN, LN)] = jnp.zeros((LN,), jnp.int32)
            vbd[r, pl.ds(k * LN, LN)] = jnp.zeros((LN,), jnp.int32)

    def chunk(c, car):
        base_e = pl.multiple_of(w * EPW + c * CH, 8)
        pltpu.sync_copy(dst_hbm.at[pl.ds(base_e, CH)], dbuf)
        pltpu.sync_copy(src_hbm.at[pl.ds(base_e, CH)], sbuf)

        for colk in range(8):
            nrow = 16 if colk <= 4 else 15

            def vec(r, car2, colk=colk):
                v = r * 8 + colk
                d = dbuf[pl.ds(v * LN, LN)]
                s = sbuf[pl.ds(v * LN, LN)]
                kd, pv = plsc.sort_key_val(d, lanes)
                sv = s.at[pv].get(mode="promise_in_bounds")
                b = _bkt(kd)
                bprev = b.at[jnp.maximum(lanes - 1, 0)].get(
                    mode="promise_in_bounds")
                mnew = (b != bprev) | (lanes == 0)
                runstart = plsc.cummax(jnp.where(mnew, lanes, -1))
                rank = lanes - runstart
                pos = plsc.load_gather(fl, [b]) + rank
                posbuf[r, pl.ds(colk * LN, LN)] = pos
                vbs[r, pl.ds(colk * LN, LN)] = sv
                vbd[r, pl.ds(colk * LN, LN)] = kd
                bnext = b.at[jnp.minimum(lanes + 1, LN - 1)].get(
                    mode="promise_in_bounds")
                mend = (b != bnext) | (lanes == LN - 1)
                plsc.store_scatter(fl, [b], pos + 1, mask=mend)
                plsc.store_scatter(stash_s, [b], sv, mask=mend)
                plsc.store_scatter(stash_d, [b], kd, mask=mend)
                return car2

            lax.fori_loop(0, nrow, vec, 0)

        for r in range(16):
            pltpu.async_copy(vbs.at[r], srcp_hbm.at[posbuf.at[r]], sem)
            pltpu.async_copy(vbd.at[r], dstp_hbm.at[posbuf.at[r]], sem)
        for r in range(16):
            pltpu.make_async_copy(
                vbs.at[r], srcp_hbm.at[posbuf.at[r]], sem).wait()
            pltpu.make_async_copy(
                vbd.at[r], dstp_hbm.at[posbuf.at[r]], sem).wait()
        return car

    lax.fori_loop(0, NCH, chunk, 0)

    for b in range(NB):
        g0 = (b // LN) * LN
        f = fl[pl.ds(g0, LN)][b % LN]
        c0 = cs0[pl.ds(g0, LN)][b % LN]
        cnt_b = f - c0
        npad = ((cnt_b + 7) & (-8)) - cnt_b
        vs = stash_s[pl.ds(g0, LN)][b % LN]
        vd = stash_d[pl.ds(g0, LN)][b % LN]

        @pl.when(npad > 0)
        def _pad(f=f, npad=npad, vs=vs, vd=vd):
            idxp[...] = f + jnp.minimum(lanes, npad - 1)
            valp_s[...] = jnp.full((LN,), vs, jnp.int32)
            valp_d[...] = jnp.full((LN,), vd, jnp.int32)
            d1 = pltpu.async_copy(valp_s, srcp_hbm.at[idxp], sem)
            d2 = pltpu.async_copy(valp_d, dstp_hbm.at[idxp], sem)
            d1.wait()
            d2.wait()


def _run_pc(src, dst, counts):
    kfn = functools.partial(
        pl.kernel, mesh=_mesh(), compiler_params=_SC_PARAMS,
        out_type=[jax.ShapeDtypeStruct((ESZ,), jnp.int32),
                  jax.ShapeDtypeStruct((ESZ,), jnp.int32)],
        scratch_types=[
            pltpu.VMEM((CH,), jnp.int32),
            pltpu.VMEM((CH,), jnp.int32),
            pltpu.VMEM((NW * NB,), jnp.int32),
            pltpu.VMEM((NW * NB,), jnp.int32),
            pltpu.VMEM((NB,), jnp.int32),
            pltpu.VMEM((NB,), jnp.int32),
            pltpu.VMEM((NB,), jnp.int32),
            pltpu.VMEM((NB,), jnp.int32),
            pltpu.VMEM((16, 128), jnp.int32),
            pltpu.VMEM((16, 128), jnp.int32),
            pltpu.VMEM((16, 128), jnp.int32),
            pltpu.VMEM((LN,), jnp.int32),
            pltpu.VMEM((LN,), jnp.int32),
            pltpu.VMEM((LN,), jnp.int32),
            pltpu.SemaphoreType.DMA,
        ],
    )(_pc_body)
    return kfn(src, dst, counts)


def _s2_body(srcp_hbm, dstp_hbm, u_hbm, v_hbm, z_hbm,
             sidx, didx, ubuf, vbuf, zbuf, sem):
    w = _wid()
    base = w * EPW2

    def do_chunk(eoff, nvalid):
        for j in range(4):
            off_j = pl.multiple_of(eoff + j * 128, 8)
            pltpu.sync_copy(srcp_hbm.at[pl.ds(off_j, 128)], sidx.at[j])
            pltpu.sync_copy(dstp_hbm.at[pl.ds(off_j, 128)], didx.at[j])
        for j in range(4):
            for k in range(8):
                iv = sidx[j, pl.ds(k * LN, LN)]
                sidx[j, pl.ds(k * LN, LN)] = jnp.minimum(
                    jnp.maximum(iv, 0), NPAD - 1)
                iv2 = didx[j, pl.ds(k * LN, LN)]
                didx[j, pl.ds(k * LN, LN)] = jnp.minimum(
                    jnp.maximum(iv2, 0), NPAD - 1)
        for j in range(4):
            pltpu.async_copy(u_hbm.at[sidx.at[j]],
                             ubuf.at[pl.ds(j * 128, 128)], sem)
            pltpu.async_copy(v_hbm.at[didx.at[j]],
                             vbuf.at[pl.ds(j * 128, 128)], sem)
        for j in range(4):
            pltpu.make_async_copy(u_hbm.at[sidx.at[j]],
                                  ubuf.at[pl.ds(j * 128, 128)], sem).wait()
            pltpu.make_async_copy(v_hbm.at[didx.at[j]],
                                  vbuf.at[pl.ds(j * 128, 128)], sem).wait()

        def cz(i, car):
            for h in range(2):
                zbuf[i, pl.ds(h * LN, LN)] = jnp.maximum(
                    ubuf[i, pl.ds(h * LN, LN)] + vbuf[i, pl.ds(h * LN, LN)],
                    0.0)
            return car

        lax.fori_loop(0, S2CB, cz, 0)
        pltpu.sync_copy(zbuf.at[pl.ds(0, nvalid)],
                        z_hbm.at[pl.ds(pl.multiple_of(eoff, 8), nvalid)])

    def chunk(c, car):
        do_chunk(base + c * S2CB, S2CB)
        return car

    lax.fori_loop(0, S2NF, chunk, 0)
    do_chunk(base + S2NF * S2CB, S2REM)


def _run_s2(srcp, dstp, u, v):
    kfn = functools.partial(
        pl.kernel, mesh=_mesh(), compiler_params=_SC_PARAMS,
        out_type=jax.ShapeDtypeStruct((EPAD, F), jnp.float32),
        scratch_types=[
            pltpu.VMEM((4, 128), jnp.int32),
            pltpu.VMEM((4, 128), jnp.int32),
            pltpu.VMEM((S2CB, F), jnp.float32),
            pltpu.VMEM((S2CB, F), jnp.float32),
            pltpu.VMEM((S2CB, F), jnp.float32),
            pltpu.SemaphoreType.DMA,
        ],
    )(_s2_body)
    return kfn(srcp, dstp, u, v)


def _s4_common(m_hbm, dstp_hbm, counts_hbm, cb, cs, mbuf, dbuf4, tbl, sem):
    w = _wid()
    pltpu.sync_copy(counts_hbm, cb)
    total = _prefix(cb, cs)
    s = plsc.load_gather(cs, [jnp.full((LN,), w, jnp.int32)])[0]
    e_next = plsc.load_gather(
        cs, [jnp.full((LN,), jnp.minimum(w + 1, NB - 1), jnp.int32)])[0]
    e = jnp.where(w == NB - 1, total, e_next)
    nodebase = w * CSP

    def zt(i, car):
        for h in range(2):
            tbl[i, pl.ds(h * LN, LN)] = jnp.zeros((LN,), jnp.float32)
        return car

    lax.fori_loop(0, CSP, zt, 0)
    dbuf4[pl.ds(0, LN)] = jnp.full((LN,), nodebase, jnp.int32)
    for r in range(8, 16):
        for h in range(2):
            mbuf[r, pl.ds(h * LN, LN)] = jnp.zeros((LN,), jnp.float32)

    def apply_grp(g):
        dv = dbuf4[pl.ds(g * LN, LN)]
        dv = jnp.minimum(jnp.maximum(dv - nodebase, 0), CSP - 1)
        for j in range(LN):
            r = dv[j]
            er = g * LN + j
            for h in range(2):
                tbl[r, pl.ds(h * LN, LN)] = jnp.maximum(
                    tbl[r, pl.ds(h * LN, LN)], mbuf[er, pl.ds(h * LN, LN)])

    cnt = e - s
    nfull = cnt >> 9

    def chunk(c, car):
        off = pl.multiple_of(s + c * S2CB, 8)
        pltpu.sync_copy(m_hbm.at[pl.ds(off, S2CB)], mbuf)
        pltpu.sync_copy(dstp_hbm.at[pl.ds(off, S2CB)], dbuf4)

        def grp(g, car2):
            apply_grp(g)
            return car2

        lax.fori_loop(0, S2CB // LN, grp, 0)
        return car

    lax.fori_loop(0, nfull, chunk, 0)
    t0 = s + (nfull << 9)
    ng8 = (e - t0) >> 3

    def g8(j, car):
        off = pl.multiple_of(t0 + j * 8, 8)
        pltpu.sync_copy(m_hbm.at[pl.ds(off, 8)], mbuf.at[pl.ds(0, 8)])
        pltpu.sync_copy(dstp_hbm.at[pl.ds(off, 8)], dbuf4.at[pl.ds(0, 8)])
        apply_grp(0)
        return car

    lax.fori_loop(0, ng8, g8, 0)
    return w, nodebase


def _s4h_body(m_hbm, dstp_hbm, counts_hbm, h_hbm,
              cb, cs, mbuf, dbuf4, tbl, sem):
    w, nodebase = _s4_common(
        m_hbm, dstp_hbm, counts_hbm, cb, cs, mbuf, dbuf4, tbl, sem)
    pltpu.sync_copy(tbl, h_hbm.at[pl.ds(pl.multiple_of(nodebase, 8), CSP)])


def _s4g_body(m_hbm, dstp_hbm, counts_hbm, batch_hbm, gpart_hbm,
              cb, cs, mbuf, dbuf4, tbl, bbuf, gtbl, sem):
    w, nodebase = _s4_common(
        m_hbm, dstp_hbm, counts_hbm, cb, cs, mbuf, dbuf4, tbl, sem)
    pltpu.sync_copy(batch_hbm.at[pl.ds(pl.multiple_of(nodebase, 8), CSP)], bbuf)
    for r in range(GG):
        for h in range(2):
            gtbl[r, pl.ds(h * LN, LN)] = jnp.zeros((LN,), jnp.float32)

    def pool(rg, car):
        bv = bbuf[pl.ds(rg * LN, LN)]
        bv = jnp.minimum(jnp.maximum(bv, 0), GG - 1)
        for j in range(LN):
            gi = bv[j]
            nr = rg * LN + j
            for h in range(2):
                gtbl[gi, pl.ds(h * LN, LN)] = jnp.maximum(
                    gtbl[gi, pl.ds(h * LN, LN)], tbl[nr, pl.ds(h * LN, LN)])
        return car

    lax.fori_loop(0, CSP // LN, pool, 0)
    pltpu.sync_copy(gtbl, gpart_hbm.at[w])


def _s4_scratch():
    return [
        pltpu.VMEM((NW * NB,), jnp.int32),
        pltpu.VMEM((NW * NB,), jnp.int32),
        pltpu.VMEM((S2CB, F), jnp.float32),
        pltpu.VMEM((S2CB,), jnp.int32),
        pltpu.VMEM((CSP, F), jnp.float32),
    ]


def _run_s4h(m, dstp, counts):
    kfn = functools.partial(
        pl.kernel, mesh=_mesh(), compiler_params=_SC_PARAMS,
        out_type=jax.ShapeDtypeStruct((NPAD, F), jnp.float32),
        scratch_types=_s4_scratch() + [pltpu.SemaphoreType.DMA],
    )(_s4h_body)
    return kfn(m, dstp, counts)


def _run_s4g(m, dstp, counts, batch_pad):
    kfn = functools.partial(
        pl.kernel, mesh=_mesh(), compiler_params=_SC_PARAMS,
        out_type=jax.ShapeDtypeStruct((NW, GG, F), jnp.float32),
        scratch_types=_s4_scratch() + [
            pltpu.VMEM((CSP,), jnp.int32),
            pltpu.VMEM((GG, F), jnp.float32),
            pltpu.SemaphoreType.DMA,
        ],
    )(_s4g_body)
    return kfn(m, dstp, counts, batch_pad)


def _s1a_body(pos_ref, w_ref, b_ref, u_ref, v_ref):
    wfull = w_ref[...]
    wh = wfull[0:3] + wfull[3:6]
    wp = wfull[3:6]
    p = pos_ref[...]
    u_ref[...] = jnp.dot(p, wh, preferred_element_type=jnp.float32) + b_ref[...]
    v_ref[...] = -jnp.dot(p, wp, preferred_element_type=jnp.float32)


def _run_s1a(pos_pad, w1, b1):
    grid = NPAD // BR1
    return pl.pallas_call(
        _s1a_body,
        grid=(grid,),
        in_specs=[
            pl.BlockSpec((BR1, 3), lambda i: (i, 0)),
            pl.BlockSpec((6, F), lambda i: (0, 0)),
            pl.BlockSpec((1, F), lambda i: (0, 0)),
        ],
        out_specs=[
            pl.BlockSpec((BR1, F), lambda i: (i, 0)),
            pl.BlockSpec((BR1, F), lambda i: (i, 0)),
        ],
        out_shape=[jax.ShapeDtypeStruct((NPAD, F), jnp.float32),
                   jax.ShapeDtypeStruct((NPAD, F), jnp.float32)],
    )(pos_pad, w1, b1)


def _s1b_body(h_ref, pos_ref, w_ref, b_ref, u_ref, v_ref):
    wfull = w_ref[...]
    wh = wfull[0:F]
    wp = wfull[F:F + 3]
    p = pos_ref[...]
    pv = jnp.dot(p, wp, preferred_element_type=jnp.float32)
    u_ref[...] = (jnp.dot(h_ref[...], wh, preferred_element_type=jnp.float32)
                  + pv + b_ref[...])
    v_ref[...] = -pv


def _run_s1b(h1, pos_pad, w1, b1):
    grid = NPAD // BR1
    return pl.pallas_call(
        _s1b_body,
        grid=(grid,),
        in_specs=[
            pl.BlockSpec((BR1, F), lambda i: (i, 0)),
            pl.BlockSpec((BR1, 3), lambda i: (i, 0)),
            pl.BlockSpec((F + 3, F), lambda i: (0, 0)),
            pl.BlockSpec((1, F), lambda i: (0, 0)),
        ],
        out_specs=[
            pl.BlockSpec((BR1, F), lambda i: (i, 0)),
            pl.BlockSpec((BR1, F), lambda i: (i, 0)),
        ],
        out_shape=[jax.ShapeDtypeStruct((NPAD, F), jnp.float32),
                   jax.ShapeDtypeStruct((NPAD, F), jnp.float32)],
    )(h1, pos_pad, w1, b1)


def _s3_body(z_ref, w_ref, b_ref, m_ref):
    m_ref[...] = (jnp.dot(z_ref[...], w_ref[...],
                          preferred_element_type=jnp.float32) + b_ref[...])


def _run_s3(z, w2, b2):
    grid = EPAD // BR3
    return pl.pallas_call(
        _s3_body,
        grid=(grid,),
        in_specs=[
            pl.BlockSpec((BR3, F), lambda i: (i, 0)),
            pl.BlockSpec((F, F), lambda i: (0, 0)),
            pl.BlockSpec((1, F), lambda i: (0, 0)),
        ],
        out_specs=pl.BlockSpec((BR3, F), lambda i: (i, 0)),
        out_shape=jax.ShapeDtypeStruct((EPAD, F), jnp.float32),
    )(z, w2, b2)


def _fin_body(gp_ref, wc_ref, bc_ref, out_ref):
    g = jnp.max(gp_ref[...], axis=0)
    out_ref[...] = (jnp.dot(g, wc_ref[...],
                            preferred_element_type=jnp.float32) + bc_ref[...])


def _run_fin(gpart, wc, bc):
    return pl.pallas_call(
        _fin_body,
        out_shape=jax.ShapeDtypeStruct((GG, wc.shape[1]), jnp.float32),
    )(gpart, wc, bc)


def kernel(pos, edge_index, batch, W1a, b1a, W2a, b2a, W1b, b1b, W2b, b2b,
           Wc, bc):
    src = edge_index[0]
    dst = edge_index[1]
    pos_pad = jnp.pad(pos, ((0, NPAD - NN), (0, 0)))
    batch_pad = jnp.pad(batch, (0, NPAD - NN))

    counts = _run_pa(dst)
    src_p, dst_p = _run_pc(src, dst, counts)

    u1, v1 = _run_s1a(pos_pad, W1a, b1a.reshape(1, F))
    z1 = _run_s2(src_p, dst_p, u1, v1)
    m1 = _run_s3(z1, W2a, b2a.reshape(1, F))
    h1 = _run_s4h(m1, dst_p, counts)

    u2, v2 = _run_s1b(h1, pos_pad, W1b, b1b.reshape(1, F))
    z2 = _run_s2(src_p, dst_p, u2, v2)
    m2 = _run_s3(z2, W2b, b2b.reshape(1, F))
    gpart = _run_s4g(m2, dst_p, counts, batch_pad)

    return _run_fin(gpart, Wc, bc.reshape(1, -1))


# speedup vs baseline: 2.4692x; 1.8492x over previous
import functools

import jax
import jax.numpy as jnp
from jax import lax
from jax.experimental import pallas as pl
from jax.experimental.pallas import tpu as pltpu
from jax.experimental.pallas import tpu_sc as plsc

NN = 100000
EE = 1600000
GG = 64
F = 32
NC, NS, LN = 2, 16, 16
NW = NC * NS
NB = 32
CSP = 3200
NPAD = NB * CSP
EPW = EE // NW
CH = 2000
NCH = EPW // CH
EPAD = 1671168
EPW2 = EPAD // NW
S2CB = 512
S2NF = EPW2 // S2CB
ESZ = EPAD
LSZ = EPW + NB * 64
BR1 = 6400
BR3 = 4096
DW = 8

_SC_PARAMS = pltpu.CompilerParams(
    needs_layout_passes=False, use_tc_tiling_on_sc=False)


def _mesh():
    return plsc.VectorSubcoreMesh(
        core_axis_name="c", subcore_axis_name="s",
        num_cores=NC, num_subcores=NS)


def _wid():
    return lax.axis_index("s") * NC + lax.axis_index("c")


def _bkt(d):
    return ((d >> 7) * 5243) >> 17


def _lanes():
    return lax.iota(jnp.int32, LN)


def _prefix(cb, cs):
    lanes = _lanes()
    carry = jnp.int32(0)
    for k in range(NW * NB // LN):
        j = k * LN + lanes
        b = j >> 5
        wv = j & 31
        idx = wv * NB + b
        c = plsc.load_gather(cb, [idx])
        cp = (c + 63) & (-64)
        excl = plsc.cumsum(cp) - cp + carry
        plsc.store_scatter(cs, [idx], excl)
        carry = carry + jnp.sum(cp)
    return carry


def _pa_body(dst_hbm, counts_hbm, dbuf, hist, cbuf, sem):
    del sem
    w = _wid()
    lanes = _lanes()
    for b in range(NB):
        hist[pl.ds(b * LN, LN)] = jnp.zeros((LN,), jnp.int32)

    def chunk(c, car):
        pltpu.sync_copy(dst_hbm.at[pl.ds(pl.multiple_of(w * EPW + c * CH, 8), CH)], dbuf)

        def hb(v, car2):
            d = dbuf[pl.ds(v * LN, LN)]
            fi = _bkt(d) * LN + lanes
            plsc.store_scatter(hist, [fi], plsc.load_gather(hist, [fi]) + 1)
            return car2

        return lax.fori_loop(0, CH // LN, hb, car)

    lax.fori_loop(0, NCH, chunk, 0)
    for g in range(NB // LN):
        acc = jnp.zeros((LN,), jnp.int32)
        for l in range(LN):
            acc = acc + plsc.load_gather(hist, [(lanes + g * LN) * LN + l])
        cbuf[pl.ds(g * LN, LN)] = acc
    pltpu.sync_copy(cbuf, counts_hbm.at[pl.ds(pl.multiple_of(w * NB, 8), NB)])


def _run_pa(dst):
    kfn = functools.partial(
        pl.kernel, mesh=_mesh(), compiler_params=_SC_PARAMS,
        out_type=jax.ShapeDtypeStruct((NW * NB,), jnp.int32),
        scratch_types=[
            pltpu.VMEM((CH,), jnp.int32),
            pltpu.VMEM((NB * LN,), jnp.int32),
            pltpu.VMEM((NB,), jnp.int32),
            pltpu.SemaphoreType.DMA,
        ],
    )(_pa_body)
    return kfn(dst)


def _pc_body(src_hbm, dst_hbm, counts_hbm, srcp_hbm, dstp_hbm,
             dbuf, sbuf, cb, cs, fl, ll0, loc_s, loc_d, sem):
    w = _wid()
    lanes = _lanes()
    pltpu.sync_copy(counts_hbm, cb)
    _prefix(cb, cs)
    lcarry = jnp.int32(0)
    for g in range(NB // LN):
        c = cb[pl.ds(w * NB + g * LN, LN)]
        cp = (c + 63) & (-64)
        excl = plsc.cumsum(cp) - cp + lcarry
        fl[pl.ds(g * LN, LN)] = excl
        ll0[pl.ds(g * LN, LN)] = excl
        lcarry = lcarry + jnp.sum(cp)

    def chunk(c, car):
        base_e = pl.multiple_of(w * EPW + c * CH, 8)
        pltpu.sync_copy(dst_hbm.at[pl.ds(base_e, CH)], dbuf)
        pltpu.sync_copy(src_hbm.at[pl.ds(base_e, CH)], sbuf)

        def vec(v, car2):
            d = dbuf[pl.ds(v * LN, LN)]
            s = sbuf[pl.ds(v * LN, LN)]
            kd, pv = plsc.sort_key_val(d, lanes)
            sv = s.at[pv].get(mode="promise_in_bounds")
            b = _bkt(kd)
            bprev = b.at[jnp.maximum(lanes - 1, 0)].get(
                mode="promise_in_bounds")
            mnew = (b != bprev) | (lanes == 0)
            runstart = plsc.cummax(jnp.where(mnew, lanes, -1))
            rank = lanes - runstart
            pos = plsc.load_gather(fl, [b]) + rank
            plsc.store_scatter(loc_s, [pos], sv)
            plsc.store_scatter(loc_d, [pos], kd)
            bnext = b.at[jnp.minimum(lanes + 1, LN - 1)].get(
                mode="promise_in_bounds")
            mend = (b != bnext) | (lanes == LN - 1)
            plsc.store_scatter(fl, [b], pos + 1, mask=mend)
            return car2

        lax.fori_loop(0, CH // LN, vec, 0)
        return car

    lax.fori_loop(0, NCH, chunk, 0)

    for b in range(NB):
        g0 = (b // LN) * LN
        f = fl[pl.ds(g0, LN)][b % LN]
        l0 = ll0[pl.ds(g0, LN)][b % LN]
        cnt_b = f - l0
        cpb = (cnt_b + 63) & (-64)
        npad = cpb - cnt_b
        lastix = jnp.maximum(f - 1, 0)
        vs = plsc.load_gather(loc_s, [jnp.full((LN,), lastix, jnp.int32)])
        vd = plsc.load_gather(loc_d, [jnp.full((LN,), lastix, jnp.int32)])
        for t in range(4):
            tl = t * LN + lanes
            idxp = f + tl
            mpad = tl < npad
            plsc.store_scatter(loc_s, [idxp], vs, mask=mpad)
            plsc.store_scatter(loc_d, [idxp], vd, mask=mpad)
        gc0 = plsc.load_gather(
            cs, [jnp.full((LN,), w * NB + b, jnp.int32)])[0]
        n64 = cpb >> 6

        def is64(j, car, l0=l0, gc0=gc0):
            so = pl.multiple_of(l0 + j * 64, 8)
            do = pl.multiple_of(gc0 + j * 64, 8)
            pltpu.async_copy(loc_s.at[pl.ds(so, 64)],
                             srcp_hbm.at[pl.ds(do, 64)], sem)
            pltpu.async_copy(loc_d.at[pl.ds(so, 64)],
                             dstp_hbm.at[pl.ds(do, 64)], sem)

            @pl.when(j >= DW)
            def _drain(j=j, l0=l0, gc0=gc0):
                sod = pl.multiple_of(l0 + (j - DW) * 64, 8)
                dod = pl.multiple_of(gc0 + (j - DW) * 64, 8)
                pltpu.make_async_copy(
                    loc_s.at[pl.ds(sod, 64)],
                    srcp_hbm.at[pl.ds(dod, 64)], sem).wait()
                pltpu.make_async_copy(
                    loc_d.at[pl.ds(sod, 64)],
                    dstp_hbm.at[pl.ds(dod, 64)], sem).wait()
            return car

        lax.fori_loop(0, n64, is64, 0)

        def drain64(j, car, l0=l0, gc0=gc0):
            sod = pl.multiple_of(l0 + j * 64, 8)
            dod = pl.multiple_of(gc0 + j * 64, 8)
            pltpu.make_async_copy(
                loc_s.at[pl.ds(sod, 64)],
                srcp_hbm.at[pl.ds(dod, 64)], sem).wait()
            pltpu.make_async_copy(
                loc_d.at[pl.ds(sod, 64)],
                dstp_hbm.at[pl.ds(dod, 64)], sem).wait()
            return car

        lax.fori_loop(jnp.maximum(n64 - DW, 0), n64, drain64, 0)


def _run_pc(src, dst, counts):
    kfn = functools.partial(
        pl.kernel, mesh=_mesh(), compiler_params=_SC_PARAMS,
        out_type=[jax.ShapeDtypeStruct((ESZ,), jnp.int32),
                  jax.ShapeDtypeStruct((ESZ,), jnp.int32)],
        scratch_types=[
            pltpu.VMEM((CH,), jnp.int32),
            pltpu.VMEM((CH,), jnp.int32),
            pltpu.VMEM((NW * NB,), jnp.int32),
            pltpu.VMEM((NW * NB,), jnp.int32),
            pltpu.VMEM((NB,), jnp.int32),
            pltpu.VMEM((NB,), jnp.int32),
            pltpu.VMEM((LSZ,), jnp.int32),
            pltpu.VMEM((LSZ,), jnp.int32),
            pltpu.SemaphoreType.DMA,
        ],
    )(_pc_body)
    return kfn(src, dst, counts)


def _s2_body(srcp_hbm, dstp_hbm, u_hbm, v_hbm, z_hbm,
             sidx, didx, ubuf, vbuf, zbuf, sem):
    w = _wid()
    base = w * EPW2

    def do_chunk(eoff, nvalid):
        for j in range(4):
            off_j = pl.multiple_of(eoff + j * 128, 8)
            pltpu.sync_copy(srcp_hbm.at[pl.ds(off_j, 128)], sidx.at[j])
            pltpu.sync_copy(dstp_hbm.at[pl.ds(off_j, 128)], didx.at[j])
        for j in range(4):
            for k in range(8):
                iv = sidx[j, pl.ds(k * LN, LN)]
                sidx[j, pl.ds(k * LN, LN)] = jnp.minimum(
                    jnp.maximum(iv, 0), NPAD - 1)
                iv2 = didx[j, pl.ds(k * LN, LN)]
                didx[j, pl.ds(k * LN, LN)] = jnp.minimum(
                    jnp.maximum(iv2, 0), NPAD - 1)
        for j in range(4):
            pltpu.async_copy(u_hbm.at[sidx.at[j]],
                             ubuf.at[pl.ds(j * 128, 128)], sem)
            pltpu.async_copy(v_hbm.at[didx.at[j]],
                             vbuf.at[pl.ds(j * 128, 128)], sem)
        for j in range(4):
            pltpu.make_async_copy(u_hbm.at[sidx.at[j]],
                                  ubuf.at[pl.ds(j * 128, 128)], sem).wait()
            pltpu.make_async_copy(v_hbm.at[didx.at[j]],
                                  vbuf.at[pl.ds(j * 128, 128)], sem).wait()

        def cz(i, car):
            for h in range(2):
                zbuf[i, pl.ds(h * LN, LN)] = jnp.maximum(
                    ubuf[i, pl.ds(h * LN, LN)] + vbuf[i, pl.ds(h * LN, LN)],
                    0.0)
            return car

        lax.fori_loop(0, S2CB, cz, 0)
        pltpu.sync_copy(zbuf.at[pl.ds(0, nvalid)],
                        z_hbm.at[pl.ds(pl.multiple_of(eoff, 8), nvalid)])

    def chunk(c, car):
        do_chunk(base + c * S2CB, S2CB)
        return car

    lax.fori_loop(0, S2NF, chunk, 0)


def _run_s2(srcp, dstp, u, v):
    kfn = functools.partial(
        pl.kernel, mesh=_mesh(), compiler_params=_SC_PARAMS,
        out_type=jax.ShapeDtypeStruct((EPAD, F), jnp.float32),
        scratch_types=[
            pltpu.VMEM((4, 128), jnp.int32),
            pltpu.VMEM((4, 128), jnp.int32),
            pltpu.VMEM((S2CB, F), jnp.float32),
            pltpu.VMEM((S2CB, F), jnp.float32),
            pltpu.VMEM((S2CB, F), jnp.float32),
            pltpu.SemaphoreType.DMA,
        ],
    )(_s2_body)
    return kfn(srcp, dstp, u, v)


def _s4_common(m_hbm, dstp_hbm, counts_hbm, cb, cs, mbuf, dbuf4, tbl, sem):
    w = _wid()
    pltpu.sync_copy(counts_hbm, cb)
    total = _prefix(cb, cs)
    s = plsc.load_gather(cs, [jnp.full((LN,), w, jnp.int32)])[0]
    e_next = plsc.load_gather(
        cs, [jnp.full((LN,), jnp.minimum(w + 1, NB - 1), jnp.int32)])[0]
    e = jnp.where(w == NB - 1, total, e_next)
    nodebase = w * CSP

    def zt(i, car):
        for h in range(2):
            tbl[i, pl.ds(h * LN, LN)] = jnp.zeros((LN,), jnp.float32)
        return car

    lax.fori_loop(0, CSP, zt, 0)
    dbuf4[pl.ds(0, LN)] = jnp.full((LN,), nodebase, jnp.int32)
    for r in range(8, 16):
        for h in range(2):
            mbuf[r, pl.ds(h * LN, LN)] = jnp.zeros((LN,), jnp.float32)

    def apply_grp(g):
        dv = dbuf4[pl.ds(g * LN, LN)]
        dv = jnp.minimum(jnp.maximum(dv - nodebase, 0), CSP - 1)
        for j in range(LN):
            r = dv[j]
            er = g * LN + j
            for h in range(2):
                tbl[r, pl.ds(h * LN, LN)] = jnp.maximum(
                    tbl[r, pl.ds(h * LN, LN)], mbuf[er, pl.ds(h * LN, LN)])

    cnt = e - s
    nfull = cnt >> 9

    def chunk(c, car):
        off = pl.multiple_of(s + c * S2CB, 8)
        pltpu.sync_copy(m_hbm.at[pl.ds(off, S2CB)], mbuf)
        pltpu.sync_copy(dstp_hbm.at[pl.ds(off, S2CB)], dbuf4)

        def grp(g, car2):
            apply_grp(g)
            return car2

        lax.fori_loop(0, S2CB // LN, grp, 0)
        return car

    lax.fori_loop(0, nfull, chunk, 0)
    t0 = s + (nfull << 9)
    ng8 = (e - t0) >> 3

    def g8(j, car):
        off = pl.multiple_of(t0 + j * 8, 8)
        pltpu.sync_copy(m_hbm.at[pl.ds(off, 8)], mbuf.at[pl.ds(0, 8)])
        pltpu.sync_copy(dstp_hbm.at[pl.ds(off, 8)], dbuf4.at[pl.ds(0, 8)])
        apply_grp(0)
        return car

    lax.fori_loop(0, ng8, g8, 0)
    return w, nodebase


def _s4h_body(m_hbm, dstp_hbm, counts_hbm, h_hbm,
              cb, cs, mbuf, dbuf4, tbl, sem):
    w, nodebase = _s4_common(
        m_hbm, dstp_hbm, counts_hbm, cb, cs, mbuf, dbuf4, tbl, sem)
    pltpu.sync_copy(tbl, h_hbm.at[pl.ds(pl.multiple_of(nodebase, 8), CSP)])


def _s4g_body(m_hbm, dstp_hbm, counts_hbm, batch_hbm, gpart_hbm,
              cb, cs, mbuf, dbuf4, tbl, bbuf, gtbl, sem):
    w, nodebase = _s4_common(
        m_hbm, dstp_hbm, counts_hbm, cb, cs, mbuf, dbuf4, tbl, sem)
    pltpu.sync_copy(batch_hbm.at[pl.ds(pl.multiple_of(nodebase, 8), CSP)], bbuf)
    for r in range(GG):
        for h in range(2):
            gtbl[r, pl.ds(h * LN, LN)] = jnp.zeros((LN,), jnp.float32)

    def pool(rg, car):
        bv = bbuf[pl.ds(rg * LN, LN)]
        bv = jnp.minimum(jnp.maximum(bv, 0), GG - 1)
        for j in range(LN):
            gi = bv[j]
            nr = rg * LN + j
            for h in range(2):
                gtbl[gi, pl.ds(h * LN, LN)] = jnp.maximum(
                    gtbl[gi, pl.ds(h * LN, LN)], tbl[nr, pl.ds(h * LN, LN)])
        return car

    lax.fori_loop(0, CSP // LN, pool, 0)
    pltpu.sync_copy(gtbl, gpart_hbm.at[w])


def _s4_scratch():
    return [
        pltpu.VMEM((NW * NB,), jnp.int32),
        pltpu.VMEM((NW * NB,), jnp.int32),
        pltpu.VMEM((S2CB, F), jnp.float32),
        pltpu.VMEM((S2CB,), jnp.int32),
        pltpu.VMEM((CSP, F), jnp.float32),
    ]


def _run_s4h(m, dstp, counts):
    kfn = functools.partial(
        pl.kernel, mesh=_mesh(), compiler_params=_SC_PARAMS,
        out_type=jax.ShapeDtypeStruct((NPAD, F), jnp.float32),
        scratch_types=_s4_scratch() + [pltpu.SemaphoreType.DMA],
    )(_s4h_body)
    return kfn(m, dstp, counts)


def _run_s4g(m, dstp, counts, batch_pad):
    kfn = functools.partial(
        pl.kernel, mesh=_mesh(), compiler_params=_SC_PARAMS,
        out_type=jax.ShapeDtypeStruct((NW, GG, F), jnp.float32),
        scratch_types=_s4_scratch() + [
            pltpu.VMEM((CSP,), jnp.int32),
            pltpu.VMEM((GG, F), jnp.float32),
            pltpu.SemaphoreType.DMA,
        ],
    )(_s4g_body)
    return kfn(m, dstp, counts, batch_pad)


def _s1a_body(pos_ref, w_ref, b_ref, u_ref, v_ref):
    wfull = w_ref[...]
    wh = wfull[0:3] + wfull[3:6]
    wp = wfull[3:6]
    p = pos_ref[...]
    u_ref[...] = jnp.dot(p, wh, preferred_element_type=jnp.float32) + b_ref[...]
    v_ref[...] = -jnp.dot(p, wp, preferred_element_type=jnp.float32)


def _run_s1a(pos_pad, w1, b1):
    grid = NPAD // BR1
    return pl.pallas_call(
        _s1a_body,
        grid=(grid,),
        in_specs=[
            pl.BlockSpec((BR1, 3), lambda i: (i, 0)),
            pl.BlockSpec((6, F), lambda i: (0, 0)),
            pl.BlockSpec((1, F), lambda i: (0, 0)),
        ],
        out_specs=[
            pl.BlockSpec((BR1, F), lambda i: (i, 0)),
            pl.BlockSpec((BR1, F), lambda i: (i, 0)),
        ],
        out_shape=[jax.ShapeDtypeStruct((NPAD, F), jnp.float32),
                   jax.ShapeDtypeStruct((NPAD, F), jnp.float32)],
    )(pos_pad, w1, b1)


def _s1b_body(h_ref, pos_ref, w_ref, b_ref, u_ref, v_ref):
    wfull = w_ref[...]
    wh = wfull[0:F]
    wp = wfull[F:F + 3]
    p = pos_ref[...]
    pv = jnp.dot(p, wp, preferred_element_type=jnp.float32)
    u_ref[...] = (jnp.dot(h_ref[...], wh, preferred_element_type=jnp.float32)
                  + pv + b_ref[...])
    v_ref[...] = -pv


def _run_s1b(h1, pos_pad, w1, b1):
    grid = NPAD // BR1
    return pl.pallas_call(
        _s1b_body,
        grid=(grid,),
        in_specs=[
            pl.BlockSpec((BR1, F), lambda i: (i, 0)),
            pl.BlockSpec((BR1, 3), lambda i: (i, 0)),
            pl.BlockSpec((F + 3, F), lambda i: (0, 0)),
            pl.BlockSpec((1, F), lambda i: (0, 0)),
        ],
        out_specs=[
            pl.BlockSpec((BR1, F), lambda i: (i, 0)),
            pl.BlockSpec((BR1, F), lambda i: (i, 0)),
        ],
        out_shape=[jax.ShapeDtypeStruct((NPAD, F), jnp.float32),
                   jax.ShapeDtypeStruct((NPAD, F), jnp.float32)],
    )(h1, pos_pad, w1, b1)


def _s3_body(z_ref, w_ref, b_ref, m_ref):
    m_ref[...] = (jnp.dot(z_ref[...], w_ref[...],
                          preferred_element_type=jnp.float32) + b_ref[...])


def _run_s3(z, w2, b2):
    grid = EPAD // BR3
    return pl.pallas_call(
        _s3_body,
        grid=(grid,),
        in_specs=[
            pl.BlockSpec((BR3, F), lambda i: (i, 0)),
            pl.BlockSpec((F, F), lambda i: (0, 0)),
            pl.BlockSpec((1, F), lambda i: (0, 0)),
        ],
        out_specs=pl.BlockSpec((BR3, F), lambda i: (i, 0)),
        out_shape=jax.ShapeDtypeStruct((EPAD, F), jnp.float32),
    )(z, w2, b2)


def _fin_body(gp_ref, wc_ref, bc_ref, out_ref):
    g = jnp.max(gp_ref[...], axis=0)
    out_ref[...] = (jnp.dot(g, wc_ref[...],
                            preferred_element_type=jnp.float32) + bc_ref[...])


def _run_fin(gpart, wc, bc):
    return pl.pallas_call(
        _fin_body,
        out_shape=jax.ShapeDtypeStruct((GG, wc.shape[1]), jnp.float32),
    )(gpart, wc, bc)


def kernel(pos, edge_index, batch, W1a, b1a, W2a, b2a, W1b, b1b, W2b, b2b,
           Wc, bc):
    src = edge_index[0]
    dst = edge_index[1]
    pos_pad = jnp.pad(pos, ((0, NPAD - NN), (0, 0)))
    batch_pad = jnp.pad(batch, (0, NPAD - NN))

    counts = _run_pa(dst)
    src_p, dst_p = _run_pc(src, dst, counts)

    u1, v1 = _run_s1a(pos_pad, W1a, b1a.reshape(1, F))
    z1 = _run_s2(src_p, dst_p, u1, v1)
    m1 = _run_s3(z1, W2a, b2a.reshape(1, F))
    h1 = _run_s4h(m1, dst_p, counts)

    u2, v2 = _run_s1b(h1, pos_pad, W1b, b1b.reshape(1, F))
    z2 = _run_s2(src_p, dst_p, u2, v2)
    m2 = _run_s3(z2, W2b, b2b.reshape(1, F))
    gpart = _run_s4g(m2, dst_p, counts, batch_pad)

    return _run_fin(gpart, Wc, bc.reshape(1, -1))


# speedup vs baseline: 3.7873x; 1.5338x over previous
import functools

import jax
import jax.numpy as jnp
from jax import lax
from jax.experimental import pallas as pl
from jax.experimental.pallas import tpu as pltpu
from jax.experimental.pallas import tpu_sc as plsc

NN = 100000
EE = 1600000
GG = 64
F = 32
NC, NS, LN = 2, 16, 16
NW = NC * NS
NB = 32
CSP = 3200
NPAD = NB * CSP
EPW = EE // NW
CH = 2000
NCH = EPW // CH
EPAD = 1671168
EPW2 = EPAD // NW
S2CB = 512
S2NF = EPW2 // S2CB
ESZ = EPAD
LSZ = EPW + NB * 64
BR1 = 6400
BR3 = 4096
EP4 = EPAD // 4
DW = 8

_SC_PARAMS = pltpu.CompilerParams(
    needs_layout_passes=False, use_tc_tiling_on_sc=False)


def _mesh():
    return plsc.VectorSubcoreMesh(
        core_axis_name="c", subcore_axis_name="s",
        num_cores=NC, num_subcores=NS)


def _wid():
    return lax.axis_index("s") * NC + lax.axis_index("c")


def _bkt(d):
    return ((d >> 7) * 5243) >> 17


def _lanes():
    return lax.iota(jnp.int32, LN)


def _prefix(cb, cs):
    lanes = _lanes()
    carry = jnp.int32(0)
    for k in range(NW * NB // LN):
        j = k * LN + lanes
        b = j >> 5
        wv = j & 31
        idx = wv * NB + b
        c = plsc.load_gather(cb, [idx])
        cp = (c + 63) & (-64)
        excl = plsc.cumsum(cp) - cp + carry
        plsc.store_scatter(cs, [idx], excl)
        carry = carry + jnp.sum(cp)
    return carry


def _pa_body(dst_hbm, counts_hbm, dbuf, hist, cbuf, sem):
    del sem
    w = _wid()
    lanes = _lanes()
    for b in range(NB):
        hist[pl.ds(b * LN, LN)] = jnp.zeros((LN,), jnp.int32)

    def chunk(c, car):
        pltpu.sync_copy(dst_hbm.at[pl.ds(pl.multiple_of(w * EPW + c * CH, 8), CH)], dbuf)

        def hb(v, car2):
            d = dbuf[pl.ds(v * LN, LN)]
            fi = _bkt(d) * LN + lanes
            plsc.store_scatter(hist, [fi], plsc.load_gather(hist, [fi]) + 1)
            return car2

        return lax.fori_loop(0, CH // LN, hb, car)

    lax.fori_loop(0, NCH, chunk, 0)
    for g in range(NB // LN):
        acc = jnp.zeros((LN,), jnp.int32)
        for l in range(LN):
            acc = acc + plsc.load_gather(hist, [(lanes + g * LN) * LN + l])
        cbuf[pl.ds(g * LN, LN)] = acc
    pltpu.sync_copy(cbuf, counts_hbm.at[pl.ds(pl.multiple_of(w * NB, 8), NB)])


def _run_pa(dst):
    kfn = functools.partial(
        pl.kernel, mesh=_mesh(), compiler_params=_SC_PARAMS,
        out_type=jax.ShapeDtypeStruct((NW * NB,), jnp.int32),
        scratch_types=[
            pltpu.VMEM((CH,), jnp.int32),
            pltpu.VMEM((NB * LN,), jnp.int32),
            pltpu.VMEM((NB,), jnp.int32),
            pltpu.SemaphoreType.DMA,
        ],
    )(_pa_body)
    return kfn(dst)


def _pc_body(src_hbm, dst_hbm, counts_hbm, srcp_hbm, dstp_hbm,
             dbuf, sbuf, cb, cs, fl, ll0, loc_s, loc_d, sem):
    w = _wid()
    lanes = _lanes()
    pltpu.sync_copy(counts_hbm, cb)
    _prefix(cb, cs)
    lcarry = jnp.int32(0)
    for g in range(NB // LN):
        c = cb[pl.ds(w * NB + g * LN, LN)]
        cp = (c + 63) & (-64)
        excl = plsc.cumsum(cp) - cp + lcarry
        fl[pl.ds(g * LN, LN)] = excl
        ll0[pl.ds(g * LN, LN)] = excl
        lcarry = lcarry + jnp.sum(cp)

    def chunk(c, car):
        base_e = pl.multiple_of(w * EPW + c * CH, 8)
        pltpu.sync_copy(dst_hbm.at[pl.ds(base_e, CH)], dbuf)
        pltpu.sync_copy(src_hbm.at[pl.ds(base_e, CH)], sbuf)

        def vec(v, car2):
            d = dbuf[pl.ds(v * LN, LN)]
            s = sbuf[pl.ds(v * LN, LN)]
            kd, pv = plsc.sort_key_val(d, lanes)
            sv = s.at[pv].get(mode="promise_in_bounds")
            b = _bkt(kd)
            bprev = b.at[jnp.maximum(lanes - 1, 0)].get(
                mode="promise_in_bounds")
            mnew = (b != bprev) | (lanes == 0)
            runstart = plsc.cummax(jnp.where(mnew, lanes, -1))
            rank = lanes - runstart
            pos = plsc.load_gather(fl, [b]) + rank
            plsc.store_scatter(loc_s, [pos], sv)
            plsc.store_scatter(loc_d, [pos], kd)
            bnext = b.at[jnp.minimum(lanes + 1, LN - 1)].get(
                mode="promise_in_bounds")
            mend = (b != bnext) | (lanes == LN - 1)
            plsc.store_scatter(fl, [b], pos + 1, mask=mend)
            return car2

        lax.fori_loop(0, CH // LN, vec, 0)
        return car

    lax.fori_loop(0, NCH, chunk, 0)

    for b in range(NB):
        g0 = (b // LN) * LN
        f = fl[pl.ds(g0, LN)][b % LN]
        l0 = ll0[pl.ds(g0, LN)][b % LN]
        cnt_b = f - l0
        cpb = (cnt_b + 63) & (-64)
        npad = cpb - cnt_b
        lastix = jnp.maximum(f - 1, 0)
        vs = plsc.load_gather(loc_s, [jnp.full((LN,), lastix, jnp.int32)])
        vd = plsc.load_gather(loc_d, [jnp.full((LN,), lastix, jnp.int32)])
        for t in range(4):
            tl = t * LN + lanes
            idxp = f + tl
            mpad = tl < npad
            plsc.store_scatter(loc_s, [idxp], vs, mask=mpad)
            plsc.store_scatter(loc_d, [idxp], vd, mask=mpad)
        gc0 = plsc.load_gather(
            cs, [jnp.full((LN,), w * NB + b, jnp.int32)])[0]
        n64 = cpb >> 6

        def is64(j, car, l0=l0, gc0=gc0):
            so = pl.multiple_of(l0 + j * 64, 8)
            do = pl.multiple_of(gc0 + j * 64, 8)
            pltpu.async_copy(loc_s.at[pl.ds(so, 64)],
                             srcp_hbm.at[pl.ds(do, 64)], sem)
            pltpu.async_copy(loc_d.at[pl.ds(so, 64)],
                             dstp_hbm.at[pl.ds(do, 64)], sem)

            @pl.when(j >= DW)
            def _drain(j=j, l0=l0, gc0=gc0):
                sod = pl.multiple_of(l0 + (j - DW) * 64, 8)
                dod = pl.multiple_of(gc0 + (j - DW) * 64, 8)
                pltpu.make_async_copy(
                    loc_s.at[pl.ds(sod, 64)],
                    srcp_hbm.at[pl.ds(dod, 64)], sem).wait()
                pltpu.make_async_copy(
                    loc_d.at[pl.ds(sod, 64)],
                    dstp_hbm.at[pl.ds(dod, 64)], sem).wait()
            return car

        lax.fori_loop(0, n64, is64, 0)

        def drain64(j, car, l0=l0, gc0=gc0):
            sod = pl.multiple_of(l0 + j * 64, 8)
            dod = pl.multiple_of(gc0 + j * 64, 8)
            pltpu.make_async_copy(
                loc_s.at[pl.ds(sod, 64)],
                srcp_hbm.at[pl.ds(dod, 64)], sem).wait()
            pltpu.make_async_copy(
                loc_d.at[pl.ds(sod, 64)],
                dstp_hbm.at[pl.ds(dod, 64)], sem).wait()
            return car

        lax.fori_loop(jnp.maximum(n64 - DW, 0), n64, drain64, 0)


def _run_pc(src, dst, counts):
    kfn = functools.partial(
        pl.kernel, mesh=_mesh(), compiler_params=_SC_PARAMS,
        out_type=[jax.ShapeDtypeStruct((ESZ,), jnp.int32),
                  jax.ShapeDtypeStruct((ESZ,), jnp.int32)],
        scratch_types=[
            pltpu.VMEM((CH,), jnp.int32),
            pltpu.VMEM((CH,), jnp.int32),
            pltpu.VMEM((NW * NB,), jnp.int32),
            pltpu.VMEM((NW * NB,), jnp.int32),
            pltpu.VMEM((NB,), jnp.int32),
            pltpu.VMEM((NB,), jnp.int32),
            pltpu.VMEM((LSZ,), jnp.int32),
            pltpu.VMEM((LSZ,), jnp.int32),
            pltpu.SemaphoreType.DMA,
        ],
    )(_pc_body)
    return kfn(src, dst, counts)


def _s2_body(srcp_hbm, dstp_hbm, u_hbm, v_hbm, z_hbm,
             sidx, didx, ubuf, vbuf, zbuf, sem):
    w = _wid()
    base = w * EPW2

    def do_chunk(eoff):
        for j in range(4):
            off_j = pl.multiple_of(eoff + j * 128, 8)
            pltpu.sync_copy(srcp_hbm.at[pl.ds(off_j, 128)], sidx.at[j])
            pltpu.sync_copy(dstp_hbm.at[pl.ds(off_j, 128)], didx.at[j])
        for j in range(4):
            for k in range(8):
                iv = sidx[j, pl.ds(k * LN, LN)]
                sidx[j, pl.ds(k * LN, LN)] = jnp.minimum(
                    jnp.maximum(iv, 0), NPAD - 1)
                iv2 = didx[j, pl.ds(k * LN, LN)]
                didx[j, pl.ds(k * LN, LN)] = jnp.minimum(
                    jnp.maximum(iv2, 0), NPAD - 1)
        for j in range(4):
            pltpu.async_copy(u_hbm.at[sidx.at[j]],
                             ubuf.at[pl.ds(j * 128, 128)], sem)
            pltpu.async_copy(v_hbm.at[didx.at[j]],
                             vbuf.at[pl.ds(j * 128, 128)], sem)
        for j in range(4):
            pltpu.make_async_copy(u_hbm.at[sidx.at[j]],
                                  ubuf.at[pl.ds(j * 128, 128)], sem).wait()
            pltpu.make_async_copy(v_hbm.at[didx.at[j]],
                                  vbuf.at[pl.ds(j * 128, 128)], sem).wait()

        def cz(i2, car):
            for q in range(4):
                for h in range(2):
                    e = i2 * 4 + q
                    zbuf[i2, pl.ds(q * 32 + h * LN, LN)] = jnp.maximum(
                        ubuf[e, pl.ds(h * LN, LN)] + vbuf[e, pl.ds(h * LN, LN)],
                        0.0)
            return car

        lax.fori_loop(0, 128, cz, 0)
        pltpu.sync_copy(
            zbuf, z_hbm.at[pl.ds(pl.multiple_of(eoff >> 2, 8), 128)])

    def chunk(c, car):
        do_chunk(base + c * S2CB)
        return car

    lax.fori_loop(0, S2NF, chunk, 0)


def _run_s2(srcp, dstp, u, v):
    kfn = functools.partial(
        pl.kernel, mesh=_mesh(), compiler_params=_SC_PARAMS,
        out_type=jax.ShapeDtypeStruct((EP4, 128), jnp.float32),
        scratch_types=[
            pltpu.VMEM((4, 128), jnp.int32),
            pltpu.VMEM((4, 128), jnp.int32),
            pltpu.VMEM((S2CB, F), jnp.float32),
            pltpu.VMEM((S2CB, F), jnp.float32),
            pltpu.VMEM((128, 128), jnp.float32),
            pltpu.SemaphoreType.DMA,
        ],
    )(_s2_body)
    return kfn(srcp, dstp, u, v)


def _s4_common(m_hbm, dstp_hbm, counts_hbm, cb, cs, mbuf, dbuf4, tbl, sem):
    w = _wid()
    pltpu.sync_copy(counts_hbm, cb)
    total = _prefix(cb, cs)
    s = plsc.load_gather(cs, [jnp.full((LN,), w, jnp.int32)])[0]
    e_next = plsc.load_gather(
        cs, [jnp.full((LN,), jnp.minimum(w + 1, NB - 1), jnp.int32)])[0]
    e = jnp.where(w == NB - 1, total, e_next)
    nodebase = w * CSP

    def zt(i, car):
        for h in range(2):
            tbl[i, pl.ds(h * LN, LN)] = jnp.zeros((LN,), jnp.float32)
        return car

    lax.fori_loop(0, CSP, zt, 0)
    dbuf4[pl.ds(0, LN)] = jnp.full((LN,), nodebase, jnp.int32)
    for r in range(4):
        for kk in range(8):
            mbuf[r, pl.ds(kk * LN, LN)] = jnp.zeros((LN,), jnp.float32)

    def apply_grp(g):
        dv = dbuf4[pl.ds(g * LN, LN)]
        dv = jnp.minimum(jnp.maximum(dv - nodebase, 0), CSP - 1)
        for j in range(LN):
            r = dv[j]
            mrow = g * 4 + (j >> 2)
            for h in range(2):
                mcol = (j & 3) * 32 + h * LN
                tbl[r, pl.ds(h * LN, LN)] = jnp.maximum(
                    tbl[r, pl.ds(h * LN, LN)], mbuf[mrow, pl.ds(mcol, LN)])

    cnt = e - s
    nfull = cnt >> 9

    def chunk(c, car):
        off = pl.multiple_of(s + c * S2CB, 8)
        pltpu.sync_copy(m_hbm.at[pl.ds(pl.multiple_of((s + c * S2CB) >> 2, 8),
                                       128)], mbuf)
        pltpu.sync_copy(dstp_hbm.at[pl.ds(off, S2CB)], dbuf4)

        def grp(g, car2):
            apply_grp(g)
            return car2

        lax.fori_loop(0, S2CB // LN, grp, 0)
        return car

    lax.fori_loop(0, nfull, chunk, 0)
    t0 = s + (nfull << 9)
    ng8 = (e - t0) >> 3

    def g8(j, car):
        off = pl.multiple_of(t0 + j * 8, 8)
        pltpu.sync_copy(m_hbm.at[pl.ds((t0 + j * 8) >> 2, 2)],
                        mbuf.at[pl.ds(0, 2)])
        pltpu.sync_copy(dstp_hbm.at[pl.ds(off, 8)], dbuf4.at[pl.ds(0, 8)])
        apply_grp(0)
        return car

    lax.fori_loop(0, ng8, g8, 0)
    return w, nodebase


def _s4h_body(m_hbm, dstp_hbm, counts_hbm, h_hbm,
              cb, cs, mbuf, dbuf4, tbl, sem):
    w, nodebase = _s4_common(
        m_hbm, dstp_hbm, counts_hbm, cb, cs, mbuf, dbuf4, tbl, sem)
    pltpu.sync_copy(tbl, h_hbm.at[pl.ds(pl.multiple_of(nodebase, 8), CSP)])


def _s4g_body(m_hbm, dstp_hbm, counts_hbm, batch_hbm, gpart_hbm,
              cb, cs, mbuf, dbuf4, tbl, bbuf, gtbl, sem):
    w, nodebase = _s4_common(
        m_hbm, dstp_hbm, counts_hbm, cb, cs, mbuf, dbuf4, tbl, sem)
    pltpu.sync_copy(batch_hbm.at[pl.ds(pl.multiple_of(nodebase, 8), CSP)], bbuf)
    for r in range(GG):
        for h in range(2):
            gtbl[r, pl.ds(h * LN, LN)] = jnp.zeros((LN,), jnp.float32)

    def pool(rg, car):
        bv = bbuf[pl.ds(rg * LN, LN)]
        bv = jnp.minimum(jnp.maximum(bv, 0), GG - 1)
        for j in range(LN):
            gi = bv[j]
            nr = rg * LN + j
            for h in range(2):
                gtbl[gi, pl.ds(h * LN, LN)] = jnp.maximum(
                    gtbl[gi, pl.ds(h * LN, LN)], tbl[nr, pl.ds(h * LN, LN)])
        return car

    lax.fori_loop(0, CSP // LN, pool, 0)
    pltpu.sync_copy(gtbl, gpart_hbm.at[w])


def _s4_scratch():
    return [
        pltpu.VMEM((NW * NB,), jnp.int32),
        pltpu.VMEM((NW * NB,), jnp.int32),
        pltpu.VMEM((128, 128), jnp.float32),
        pltpu.VMEM((S2CB,), jnp.int32),
        pltpu.VMEM((CSP, F), jnp.float32),
    ]


def _run_s4h(m, dstp, counts):
    kfn = functools.partial(
        pl.kernel, mesh=_mesh(), compiler_params=_SC_PARAMS,
        out_type=jax.ShapeDtypeStruct((NPAD, F), jnp.float32),
        scratch_types=_s4_scratch() + [pltpu.SemaphoreType.DMA],
    )(_s4h_body)
    return kfn(m, dstp, counts)


def _run_s4g(m, dstp, counts, batch_pad):
    kfn = functools.partial(
        pl.kernel, mesh=_mesh(), compiler_params=_SC_PARAMS,
        out_type=jax.ShapeDtypeStruct((NW, GG, F), jnp.float32),
        scratch_types=_s4_scratch() + [
            pltpu.VMEM((CSP,), jnp.int32),
            pltpu.VMEM((GG, F), jnp.float32),
            pltpu.SemaphoreType.DMA,
        ],
    )(_s4g_body)
    return kfn(m, dstp, counts, batch_pad)


def _s1a_body(pos_ref, w_ref, b_ref, u_ref, v_ref):
    wfull = w_ref[...]
    wh = wfull[0:3] + wfull[3:6]
    wp = wfull[3:6]
    p = pos_ref[...]
    u_ref[...] = jnp.dot(p, wh, preferred_element_type=jnp.float32) + b_ref[...]
    v_ref[...] = -jnp.dot(p, wp, preferred_element_type=jnp.float32)


def _run_s1a(pos_pad, w1, b1):
    grid = NPAD // BR1
    return pl.pallas_call(
        _s1a_body,
        grid=(grid,),
        in_specs=[
            pl.BlockSpec((BR1, 3), lambda i: (i, 0)),
            pl.BlockSpec((6, F), lambda i: (0, 0)),
            pl.BlockSpec((1, F), lambda i: (0, 0)),
        ],
        out_specs=[
            pl.BlockSpec((BR1, F), lambda i: (i, 0)),
            pl.BlockSpec((BR1, F), lambda i: (i, 0)),
        ],
        out_shape=[jax.ShapeDtypeStruct((NPAD, F), jnp.float32),
                   jax.ShapeDtypeStruct((NPAD, F), jnp.float32)],
    )(pos_pad, w1, b1)


def _s1b_body(h_ref, pos_ref, w_ref, b_ref, u_ref, v_ref):
    wfull = w_ref[...]
    wh = wfull[0:F]
    wp = wfull[F:F + 3]
    p = pos_ref[...]
    pv = jnp.dot(p, wp, preferred_element_type=jnp.float32)
    u_ref[...] = (jnp.dot(h_ref[...], wh, preferred_element_type=jnp.float32)
                  + pv + b_ref[...])
    v_ref[...] = -pv


def _run_s1b(h1, pos_pad, w1, b1):
    grid = NPAD // BR1
    return pl.pallas_call(
        _s1b_body,
        grid=(grid,),
        in_specs=[
            pl.BlockSpec((BR1, F), lambda i: (i, 0)),
            pl.BlockSpec((BR1, 3), lambda i: (i, 0)),
            pl.BlockSpec((F + 3, F), lambda i: (0, 0)),
            pl.BlockSpec((1, F), lambda i: (0, 0)),
        ],
        out_specs=[
            pl.BlockSpec((BR1, F), lambda i: (i, 0)),
            pl.BlockSpec((BR1, F), lambda i: (i, 0)),
        ],
        out_shape=[jax.ShapeDtypeStruct((NPAD, F), jnp.float32),
                   jax.ShapeDtypeStruct((NPAD, F), jnp.float32)],
    )(h1, pos_pad, w1, b1)


def _s3_body(z_ref, w_ref, b_ref, m_ref):
    m_ref[...] = (jnp.dot(z_ref[...], w_ref[...],
                          preferred_element_type=jnp.float32) + b_ref[...])


def _run_s3(z, w2bd, b2t):
    grid = EP4 // (BR3 // 4)
    return pl.pallas_call(
        _s3_body,
        grid=(grid,),
        in_specs=[
            pl.BlockSpec((BR3 // 4, 128), lambda i: (i, 0)),
            pl.BlockSpec((128, 128), lambda i: (0, 0)),
            pl.BlockSpec((1, 128), lambda i: (0, 0)),
        ],
        out_specs=pl.BlockSpec((BR3 // 4, 128), lambda i: (i, 0)),
        out_shape=jax.ShapeDtypeStruct((EP4, 128), jnp.float32),
    )(z, w2bd, b2t)


def _fin_body(gp_ref, wc_ref, bc_ref, out_ref):
    g = jnp.max(gp_ref[...], axis=0)
    out_ref[...] = (jnp.dot(g, wc_ref[...],
                            preferred_element_type=jnp.float32) + bc_ref[...])


def _run_fin(gpart, wc, bc):
    return pl.pallas_call(
        _fin_body,
        out_shape=jax.ShapeDtypeStruct((GG, wc.shape[1]), jnp.float32),
    )(gpart, wc, bc)


def kernel(pos, edge_index, batch, W1a, b1a, W2a, b2a, W1b, b1b, W2b, b2b,
           Wc, bc):
    src = edge_index[0]
    dst = edge_index[1]
    pos_pad = jnp.pad(pos, ((0, NPAD - NN), (0, 0)))
    batch_pad = jnp.pad(batch, (0, NPAD - NN))

    zf = jnp.zeros((F, F), jnp.float32)
    w2bd_a = jnp.block([[W2a, zf, zf, zf], [zf, W2a, zf, zf],
                        [zf, zf, W2a, zf], [zf, zf, zf, W2a]])
    w2bd_b = jnp.block([[W2b, zf, zf, zf], [zf, W2b, zf, zf],
                        [zf, zf, W2b, zf], [zf, zf, zf, W2b]])
    b2t_a = jnp.tile(b2a, 4).reshape(1, 128)
    b2t_b = jnp.tile(b2b, 4).reshape(1, 128)

    counts = _run_pa(dst)
    src_p, dst_p = _run_pc(src, dst, counts)

    u1, v1 = _run_s1a(pos_pad, W1a, b1a.reshape(1, F))
    z1 = _run_s2(src_p, dst_p, u1, v1)
    m1 = _run_s3(z1, w2bd_a, b2t_a)
    h1 = _run_s4h(m1, dst_p, counts)

    u2, v2 = _run_s1b(h1, pos_pad, W1b, b1b.reshape(1, F))
    z2 = _run_s2(src_p, dst_p, u2, v2)
    m2 = _run_s3(z2, w2bd_b, b2t_b)
    gpart = _run_s4g(m2, dst_p, counts, batch_pad)

    return _run_fin(gpart, Wc, bc.reshape(1, -1))


# speedup vs baseline: 4.7565x; 1.2559x over previous
import functools

import jax
import jax.numpy as jnp
from jax import lax
from jax.experimental import pallas as pl
from jax.experimental.pallas import tpu as pltpu
from jax.experimental.pallas import tpu_sc as plsc

NN = 100000
EE = 1600000
GG = 64
F = 32
NC, NS, LN = 2, 16, 16
NW = NC * NS
NB = 32
CSP = 3200
NPAD = NB * CSP
EPW = EE // NW
CH = 2000
NCH = EPW // CH
EPAD = 1671168
EPW2 = EPAD // NW
S2CB = 512
S2NF = EPW2 // S2CB
ESZ = EPAD
LSZ = EPW + NB * 64
BR1 = 6400
BR3 = 4096
EP4 = EPAD // 4
DW = 8

_SC_PARAMS = pltpu.CompilerParams(
    needs_layout_passes=False, use_tc_tiling_on_sc=False)


def _mesh():
    return plsc.VectorSubcoreMesh(
        core_axis_name="c", subcore_axis_name="s",
        num_cores=NC, num_subcores=NS)


def _wid():
    return lax.axis_index("s") * NC + lax.axis_index("c")


def _bkt(d):
    return ((d >> 7) * 5243) >> 17


def _lanes():
    return lax.iota(jnp.int32, LN)


def _prefix(cb, cs):
    lanes = _lanes()
    carry = jnp.int32(0)
    for k in range(NW * NB // LN):
        j = k * LN + lanes
        b = j >> 5
        wv = j & 31
        idx = wv * NB + b
        c = plsc.load_gather(cb, [idx])
        cp = (c + 63) & (-64)
        excl = plsc.cumsum(cp) - cp + carry
        plsc.store_scatter(cs, [idx], excl)
        carry = carry + jnp.sum(cp)
    return carry


def _pa_body(dst_hbm, counts_hbm, dbuf, hist, cbuf, sem):
    del sem
    w = _wid()
    lanes = _lanes()
    for b in range(NB):
        hist[pl.ds(b * LN, LN)] = jnp.zeros((LN,), jnp.int32)

    def chunk(c, car):
        pltpu.sync_copy(dst_hbm.at[pl.ds(pl.multiple_of(w * EPW + c * CH, 8), CH)], dbuf)

        def hb(v, car2):
            d = dbuf[pl.ds(v * LN, LN)]
            fi = _bkt(d) * LN + lanes
            plsc.store_scatter(hist, [fi], plsc.load_gather(hist, [fi]) + 1)
            return car2

        return lax.fori_loop(0, CH // LN, hb, car)

    lax.fori_loop(0, NCH, chunk, 0)
    for g in range(NB // LN):
        acc = jnp.zeros((LN,), jnp.int32)
        for l in range(LN):
            acc = acc + plsc.load_gather(hist, [(lanes + g * LN) * LN + l])
        cbuf[pl.ds(g * LN, LN)] = acc
    pltpu.sync_copy(cbuf, counts_hbm.at[pl.ds(pl.multiple_of(w * NB, 8), NB)])


def _run_pa(dst):
    kfn = functools.partial(
        pl.kernel, mesh=_mesh(), compiler_params=_SC_PARAMS,
        out_type=jax.ShapeDtypeStruct((NW * NB,), jnp.int32),
        scratch_types=[
            pltpu.VMEM((CH,), jnp.int32),
            pltpu.VMEM((NB * LN,), jnp.int32),
            pltpu.VMEM((NB,), jnp.int32),
            pltpu.SemaphoreType.DMA,
        ],
    )(_pa_body)
    return kfn(dst)


def _pc_body(src_hbm, dst_hbm, counts_hbm, srcp_hbm, dstp_hbm,
             dbuf, sbuf, cb, cs, fl, ll0, loc_s, loc_d, sem):
    w = _wid()
    lanes = _lanes()
    pltpu.sync_copy(counts_hbm, cb)
    _prefix(cb, cs)
    lcarry = jnp.int32(0)
    for g in range(NB // LN):
        c = cb[pl.ds(w * NB + g * LN, LN)]
        cp = (c + 63) & (-64)
        excl = plsc.cumsum(cp) - cp + lcarry
        fl[pl.ds(g * LN, LN)] = excl
        ll0[pl.ds(g * LN, LN)] = excl
        lcarry = lcarry + jnp.sum(cp)

    def chunk(c, car):
        base_e = pl.multiple_of(w * EPW + c * CH, 8)
        pltpu.sync_copy(dst_hbm.at[pl.ds(base_e, CH)], dbuf)
        pltpu.sync_copy(src_hbm.at[pl.ds(base_e, CH)], sbuf)

        def vec(v, car2):
            d = dbuf[pl.ds(v * LN, LN)]
            s = sbuf[pl.ds(v * LN, LN)]
            kd, pv = plsc.sort_key_val(d, lanes)
            sv = s.at[pv].get(mode="promise_in_bounds")
            b = _bkt(kd)
            bprev = b.at[jnp.maximum(lanes - 1, 0)].get(
                mode="promise_in_bounds")
            mnew = (b != bprev) | (lanes == 0)
            runstart = plsc.cummax(jnp.where(mnew, lanes, -1))
            rank = lanes - runstart
            pos = plsc.load_gather(fl, [b]) + rank
            plsc.store_scatter(loc_s, [pos], sv)
            plsc.store_scatter(loc_d, [pos], kd)
            bnext = b.at[jnp.minimum(lanes + 1, LN - 1)].get(
                mode="promise_in_bounds")
            mend = (b != bnext) | (lanes == LN - 1)
            plsc.store_scatter(fl, [b], pos + 1, mask=mend)
            return car2

        lax.fori_loop(0, CH // LN, vec, 0)
        return car

    lax.fori_loop(0, NCH, chunk, 0)

    for b in range(NB):
        g0 = (b // LN) * LN
        f = fl[pl.ds(g0, LN)][b % LN]
        l0 = ll0[pl.ds(g0, LN)][b % LN]
        cnt_b = f - l0
        cpb = (cnt_b + 63) & (-64)
        npad = cpb - cnt_b
        lastix = jnp.maximum(f - 1, 0)
        vs = plsc.load_gather(loc_s, [jnp.full((LN,), lastix, jnp.int32)])
        vd = plsc.load_gather(loc_d, [jnp.full((LN,), lastix, jnp.int32)])
        for t in range(4):
            tl = t * LN + lanes
            idxp = f + tl
            mpad = tl < npad
            plsc.store_scatter(loc_s, [idxp], vs, mask=mpad)
            plsc.store_scatter(loc_d, [idxp], vd, mask=mpad)
        gc0 = plsc.load_gather(
            cs, [jnp.full((LN,), w * NB + b, jnp.int32)])[0]
        n64 = cpb >> 6

        def is64(j, car, l0=l0, gc0=gc0):
            so = pl.multiple_of(l0 + j * 64, 8)
            do = pl.multiple_of(gc0 + j * 64, 8)
            pltpu.async_copy(loc_s.at[pl.ds(so, 64)],
                             srcp_hbm.at[pl.ds(do, 64)], sem)
            pltpu.async_copy(loc_d.at[pl.ds(so, 64)],
                             dstp_hbm.at[pl.ds(do, 64)], sem)

            @pl.when(j >= DW)
            def _drain(j=j, l0=l0, gc0=gc0):
                sod = pl.multiple_of(l0 + (j - DW) * 64, 8)
                dod = pl.multiple_of(gc0 + (j - DW) * 64, 8)
                pltpu.make_async_copy(
                    loc_s.at[pl.ds(sod, 64)],
                    srcp_hbm.at[pl.ds(dod, 64)], sem).wait()
                pltpu.make_async_copy(
                    loc_d.at[pl.ds(sod, 64)],
                    dstp_hbm.at[pl.ds(dod, 64)], sem).wait()
            return car

        lax.fori_loop(0, n64, is64, 0)

        def drain64(j, car, l0=l0, gc0=gc0):
            sod = pl.multiple_of(l0 + j * 64, 8)
            dod = pl.multiple_of(gc0 + j * 64, 8)
            pltpu.make_async_copy(
                loc_s.at[pl.ds(sod, 64)],
                srcp_hbm.at[pl.ds(dod, 64)], sem).wait()
            pltpu.make_async_copy(
                loc_d.at[pl.ds(sod, 64)],
                dstp_hbm.at[pl.ds(dod, 64)], sem).wait()
            return car

        lax.fori_loop(jnp.maximum(n64 - DW, 0), n64, drain64, 0)


def _run_pc(src, dst, counts):
    kfn = functools.partial(
        pl.kernel, mesh=_mesh(), compiler_params=_SC_PARAMS,
        out_type=[jax.ShapeDtypeStruct((ESZ,), jnp.int32),
                  jax.ShapeDtypeStruct((ESZ,), jnp.int32)],
        scratch_types=[
            pltpu.VMEM((CH,), jnp.int32),
            pltpu.VMEM((CH,), jnp.int32),
            pltpu.VMEM((NW * NB,), jnp.int32),
            pltpu.VMEM((NW * NB,), jnp.int32),
            pltpu.VMEM((NB,), jnp.int32),
            pltpu.VMEM((NB,), jnp.int32),
            pltpu.VMEM((LSZ,), jnp.int32),
            pltpu.VMEM((LSZ,), jnp.int32),
            pltpu.SemaphoreType.DMA,
        ],
    )(_pc_body)
    return kfn(src, dst, counts)


def _s2_body(srcp_hbm, dstp_hbm, u_hbm, v_hbm, z_hbm,
             sidx, didx, ubuf, vbuf, zbuf, isem, gsem_a, gsem_b):
    w = _wid()
    base = w * EPW2
    gsems = (gsem_a, gsem_b)

    def prefetch(eoff, p):
        for j in range(4):
            off_j = pl.multiple_of(eoff + j * 128, 8)
            pltpu.async_copy(srcp_hbm.at[pl.ds(off_j, 128)],
                             sidx.at[p * 4 + j], isem)
            pltpu.async_copy(dstp_hbm.at[pl.ds(off_j, 128)],
                             didx.at[p * 4 + j], isem)
        for j in range(4):
            off_j = pl.multiple_of(eoff + j * 128, 8)
            pltpu.make_async_copy(srcp_hbm.at[pl.ds(off_j, 128)],
                                  sidx.at[p * 4 + j], isem).wait()
            pltpu.make_async_copy(dstp_hbm.at[pl.ds(off_j, 128)],
                                  didx.at[p * 4 + j], isem).wait()
        for j in range(4):
            for k in range(8):
                iv = sidx[p * 4 + j, pl.ds(k * LN, LN)]
                sidx[p * 4 + j, pl.ds(k * LN, LN)] = jnp.minimum(
                    jnp.maximum(iv, 0), NPAD - 1)
                iv2 = didx[p * 4 + j, pl.ds(k * LN, LN)]
                didx[p * 4 + j, pl.ds(k * LN, LN)] = jnp.minimum(
                    jnp.maximum(iv2, 0), NPAD - 1)
        for j in range(4):
            pltpu.async_copy(u_hbm.at[sidx.at[p * 4 + j]],
                             ubuf.at[pl.ds(p * 512 + j * 128, 128)], gsems[p])
            pltpu.async_copy(v_hbm.at[didx.at[p * 4 + j]],
                             vbuf.at[pl.ds(p * 512 + j * 128, 128)], gsems[p])

    def consume(eoff, p):
        for j in range(4):
            pltpu.make_async_copy(
                u_hbm.at[sidx.at[p * 4 + j]],
                ubuf.at[pl.ds(p * 512 + j * 128, 128)], gsems[p]).wait()
            pltpu.make_async_copy(
                v_hbm.at[didx.at[p * 4 + j]],
                vbuf.at[pl.ds(p * 512 + j * 128, 128)], gsems[p]).wait()

        def cz(i2, car):
            for q in range(4):
                for h in range(2):
                    e = p * 512 + i2 * 4 + q
                    zbuf[i2, pl.ds(q * 32 + h * LN, LN)] = jnp.maximum(
                        ubuf[e, pl.ds(h * LN, LN)] + vbuf[e, pl.ds(h * LN, LN)],
                        0.0)
            return car

        lax.fori_loop(0, 128, cz, 0)
        pltpu.sync_copy(
            zbuf, z_hbm.at[pl.ds(pl.multiple_of(eoff >> 2, 8), 128)])

    prefetch(base, 0)

    def loop(c2, car):
        e0 = base + (2 * c2) * S2CB
        e1 = base + (2 * c2 + 1) * S2CB
        prefetch(e1, 1)
        consume(e0, 0)

        @pl.when(c2 < S2NF // 2 - 1)
        def _pf():
            prefetch(e1 + S2CB, 0)
        consume(e1, 1)
        return car

    lax.fori_loop(0, S2NF // 2, loop, 0)


def _run_s2(srcp, dstp, u, v):
    kfn = functools.partial(
        pl.kernel, mesh=_mesh(), compiler_params=_SC_PARAMS,
        out_type=jax.ShapeDtypeStruct((EP4, 128), jnp.float32),
        scratch_types=[
            pltpu.VMEM((8, 128), jnp.int32),
            pltpu.VMEM((8, 128), jnp.int32),
            pltpu.VMEM((1024, F), jnp.float32),
            pltpu.VMEM((1024, F), jnp.float32),
            pltpu.VMEM((128, 128), jnp.float32),
            pltpu.SemaphoreType.DMA,
            pltpu.SemaphoreType.DMA,
            pltpu.SemaphoreType.DMA,
        ],
    )(_s2_body)
    return kfn(srcp, dstp, u, v)


def _s4_common(m_hbm, dstp_hbm, counts_hbm, cb, cs, mbuf, dbuf4, tbl, sem):
    w = _wid()
    pltpu.sync_copy(counts_hbm, cb)
    total = _prefix(cb, cs)
    s = plsc.load_gather(cs, [jnp.full((LN,), w, jnp.int32)])[0]
    e_next = plsc.load_gather(
        cs, [jnp.full((LN,), jnp.minimum(w + 1, NB - 1), jnp.int32)])[0]
    e = jnp.where(w == NB - 1, total, e_next)
    nodebase = w * CSP

    def zt(i, car):
        for h in range(2):
            tbl[i, pl.ds(h * LN, LN)] = jnp.zeros((LN,), jnp.float32)
        return car

    lax.fori_loop(0, CSP, zt, 0)
    dbuf4[pl.ds(0, LN)] = jnp.full((LN,), nodebase, jnp.int32)
    for r in range(4):
        for kk in range(8):
            mbuf[r, pl.ds(kk * LN, LN)] = jnp.zeros((LN,), jnp.float32)

    def apply_grp(g, p):
        dv = dbuf4[pl.ds(p * 256 + g * LN, LN)]
        dv = jnp.minimum(jnp.maximum(dv - nodebase, 0), CSP - 1)
        for j in range(LN):
            r = dv[j]
            mrow = p * 64 + g * 4 + (j >> 2)
            for h in range(2):
                mcol = (j & 3) * 32 + h * LN
                tbl[r, pl.ds(h * LN, LN)] = jnp.maximum(
                    tbl[r, pl.ds(h * LN, LN)], mbuf[mrow, pl.ds(mcol, LN)])

    cnt = e - s
    nfull = cnt >> 8

    def fire(c, p):
        off = pl.multiple_of(s + c * 256, 8)
        moff = pl.multiple_of((s + c * 256) >> 2, 8)
        pltpu.async_copy(m_hbm.at[pl.ds(moff, 64)],
                         mbuf.at[pl.ds(p * 64, 64)], sem)
        pltpu.async_copy(dstp_hbm.at[pl.ds(off, 256)],
                         dbuf4.at[pl.ds(p * 256, 256)], sem)

    def drain(c, p):
        off = pl.multiple_of(s + c * 256, 8)
        moff = pl.multiple_of((s + c * 256) >> 2, 8)
        pltpu.make_async_copy(m_hbm.at[pl.ds(moff, 64)],
                              mbuf.at[pl.ds(p * 64, 64)], sem).wait()
        pltpu.make_async_copy(dstp_hbm.at[pl.ds(off, 256)],
                              dbuf4.at[pl.ds(p * 256, 256)], sem).wait()

    @pl.when(nfull > 0)
    def _p0():
        fire(0, 0)

    def chunk(c, car):
        p = c & 1
        drain(c, p)

        @pl.when(c + 1 < nfull)
        def _pf(c=c, p=p):
            fire(c + 1, 1 - p)

        def grp(g, car2):
            apply_grp(g, p)
            return car2

        lax.fori_loop(0, 256 // LN, grp, 0)
        return car

    lax.fori_loop(0, nfull, chunk, 0)
    t0 = s + (nfull << 8)
    ng8 = (e - t0) >> 3

    def g8(j, car):
        off = pl.multiple_of(t0 + j * 8, 8)
        pltpu.sync_copy(m_hbm.at[pl.ds((t0 + j * 8) >> 2, 2)],
                        mbuf.at[pl.ds(0, 2)])
        pltpu.sync_copy(dstp_hbm.at[pl.ds(off, 8)], dbuf4.at[pl.ds(0, 8)])
        apply_grp(0, 0)
        return car

    lax.fori_loop(0, ng8, g8, 0)
    return w, nodebase


def _s4h_body(m_hbm, dstp_hbm, counts_hbm, h_hbm,
              cb, cs, mbuf, dbuf4, tbl, sem):
    w, nodebase = _s4_common(
        m_hbm, dstp_hbm, counts_hbm, cb, cs, mbuf, dbuf4, tbl, sem)
    pltpu.sync_copy(tbl, h_hbm.at[pl.ds(pl.multiple_of(nodebase, 8), CSP)])


def _s4g_body(m_hbm, dstp_hbm, counts_hbm, batch_hbm, gpart_hbm,
              cb, cs, mbuf, dbuf4, tbl, bbuf, gtbl, sem):
    w, nodebase = _s4_common(
        m_hbm, dstp_hbm, counts_hbm, cb, cs, mbuf, dbuf4, tbl, sem)
    pltpu.sync_copy(batch_hbm.at[pl.ds(pl.multiple_of(nodebase, 8), CSP)], bbuf)
    for r in range(GG):
        for h in range(2):
            gtbl[r, pl.ds(h * LN, LN)] = jnp.zeros((LN,), jnp.float32)

    def pool(rg, car):
        bv = bbuf[pl.ds(rg * LN, LN)]
        bv = jnp.minimum(jnp.maximum(bv, 0), GG - 1)
        for j in range(LN):
            gi = bv[j]
            nr = rg * LN + j
            for h in range(2):
                gtbl[gi, pl.ds(h * LN, LN)] = jnp.maximum(
                    gtbl[gi, pl.ds(h * LN, LN)], tbl[nr, pl.ds(h * LN, LN)])
        return car

    lax.fori_loop(0, CSP // LN, pool, 0)
    pltpu.sync_copy(gtbl, gpart_hbm.at[w])


def _s4_scratch():
    return [
        pltpu.VMEM((NW * NB,), jnp.int32),
        pltpu.VMEM((NW * NB,), jnp.int32),
        pltpu.VMEM((128, 128), jnp.float32),
        pltpu.VMEM((512,), jnp.int32),
        pltpu.VMEM((CSP, F), jnp.float32),
    ]


def _run_s4h(m, dstp, counts):
    kfn = functools.partial(
        pl.kernel, mesh=_mesh(), compiler_params=_SC_PARAMS,
        out_type=jax.ShapeDtypeStruct((NPAD, F), jnp.float32),
        scratch_types=_s4_scratch() + [pltpu.SemaphoreType.DMA],
    )(_s4h_body)
    return kfn(m, dstp, counts)


def _run_s4g(m, dstp, counts, batch_pad):
    kfn = functools.partial(
        pl.kernel, mesh=_mesh(), compiler_params=_SC_PARAMS,
        out_type=jax.ShapeDtypeStruct((NW, GG, F), jnp.float32),
        scratch_types=_s4_scratch() + [
            pltpu.VMEM((CSP,), jnp.int32),
            pltpu.VMEM((GG, F), jnp.float32),
            pltpu.SemaphoreType.DMA,
        ],
    )(_s4g_body)
    return kfn(m, dstp, counts, batch_pad)


def _s1a_body(pos_ref, w_ref, b_ref, u_ref, v_ref):
    wfull = w_ref[...]
    wh = wfull[0:3] + wfull[3:6]
    wp = wfull[3:6]
    p = pos_ref[...]
    u_ref[...] = jnp.dot(p, wh, preferred_element_type=jnp.float32) + b_ref[...]
    v_ref[...] = -jnp.dot(p, wp, preferred_element_type=jnp.float32)


def _run_s1a(pos_pad, w1, b1):
    grid = NPAD // BR1
    return pl.pallas_call(
        _s1a_body,
        grid=(grid,),
        in_specs=[
            pl.BlockSpec((BR1, 3), lambda i: (i, 0)),
            pl.BlockSpec((6, F), lambda i: (0, 0)),
            pl.BlockSpec((1, F), lambda i: (0, 0)),
        ],
        out_specs=[
            pl.BlockSpec((BR1, F), lambda i: (i, 0)),
            pl.BlockSpec((BR1, F), lambda i: (i, 0)),
        ],
        out_shape=[jax.ShapeDtypeStruct((NPAD, F), jnp.float32),
                   jax.ShapeDtypeStruct((NPAD, F), jnp.float32)],
    )(pos_pad, w1, b1)


def _s1b_body(h_ref, pos_ref, w_ref, b_ref, u_ref, v_ref):
    wfull = w_ref[...]
    wh = wfull[0:F]
    wp = wfull[F:F + 3]
    p = pos_ref[...]
    pv = jnp.dot(p, wp, preferred_element_type=jnp.float32)
    u_ref[...] = (jnp.dot(h_ref[...], wh, preferred_element_type=jnp.float32)
                  + pv + b_ref[...])
    v_ref[...] = -pv


def _run_s1b(h1, pos_pad, w1, b1):
    grid = NPAD // BR1
    return pl.pallas_call(
        _s1b_body,
        grid=(grid,),
        in_specs=[
            pl.BlockSpec((BR1, F), lambda i: (i, 0)),
            pl.BlockSpec((BR1, 3), lambda i: (i, 0)),
            pl.BlockSpec((F + 3, F), lambda i: (0, 0)),
            pl.BlockSpec((1, F), lambda i: (0, 0)),
        ],
        out_specs=[
            pl.BlockSpec((BR1, F), lambda i: (i, 0)),
            pl.BlockSpec((BR1, F), lambda i: (i, 0)),
        ],
        out_shape=[jax.ShapeDtypeStruct((NPAD, F), jnp.float32),
                   jax.ShapeDtypeStruct((NPAD, F), jnp.float32)],
    )(h1, pos_pad, w1, b1)


def _s3_body(z_ref, w_ref, b_ref, m_ref):
    m_ref[...] = (jnp.dot(z_ref[...], w_ref[...],
                          preferred_element_type=jnp.float32) + b_ref[...])


def _run_s3(z, w2bd, b2t):
    grid = EP4 // (BR3 // 4)
    return pl.pallas_call(
        _s3_body,
        grid=(grid,),
        in_specs=[
            pl.BlockSpec((BR3 // 4, 128), lambda i: (i, 0)),
            pl.BlockSpec((128, 128), lambda i: (0, 0)),
            pl.BlockSpec((1, 128), lambda i: (0, 0)),
        ],
        out_specs=pl.BlockSpec((BR3 // 4, 128), lambda i: (i, 0)),
        out_shape=jax.ShapeDtypeStruct((EP4, 128), jnp.float32),
    )(z, w2bd, b2t)


def _fin_body(gp_ref, wc_ref, bc_ref, out_ref):
    g = jnp.max(gp_ref[...], axis=0)
    out_ref[...] = (jnp.dot(g, wc_ref[...],
                            preferred_element_type=jnp.float32) + bc_ref[...])


def _run_fin(gpart, wc, bc):
    return pl.pallas_call(
        _fin_body,
        out_shape=jax.ShapeDtypeStruct((GG, wc.shape[1]), jnp.float32),
    )(gpart, wc, bc)


def kernel(pos, edge_index, batch, W1a, b1a, W2a, b2a, W1b, b1b, W2b, b2b,
           Wc, bc):
    src = edge_index[0]
    dst = edge_index[1]
    pos_pad = jnp.pad(pos, ((0, NPAD - NN), (0, 0)))
    batch_pad = jnp.pad(batch, (0, NPAD - NN))

    zf = jnp.zeros((F, F), jnp.float32)
    w2bd_a = jnp.block([[W2a, zf, zf, zf], [zf, W2a, zf, zf],
                        [zf, zf, W2a, zf], [zf, zf, zf, W2a]])
    w2bd_b = jnp.block([[W2b, zf, zf, zf], [zf, W2b, zf, zf],
                        [zf, zf, W2b, zf], [zf, zf, zf, W2b]])
    b2t_a = jnp.tile(b2a, 4).reshape(1, 128)
    b2t_b = jnp.tile(b2b, 4).reshape(1, 128)

    counts = _run_pa(dst)
    src_p, dst_p = _run_pc(src, dst, counts)

    u1, v1 = _run_s1a(pos_pad, W1a, b1a.reshape(1, F))
    z1 = _run_s2(src_p, dst_p, u1, v1)
    m1 = _run_s3(z1, w2bd_a, b2t_a)
    h1 = _run_s4h(m1, dst_p, counts)

    u2, v2 = _run_s1b(h1, pos_pad, W1b, b1b.reshape(1, F))
    z2 = _run_s2(src_p, dst_p, u2, v2)
    m2 = _run_s3(z2, w2bd_b, b2t_b)
    gpart = _run_s4g(m2, dst_p, counts, batch_pad)

    return _run_fin(gpart, Wc, bc.reshape(1, -1))


# speedup vs baseline: 4.9837x; 1.0477x over previous
import functools

import jax
import jax.numpy as jnp
from jax import lax
from jax.experimental import pallas as pl
from jax.experimental.pallas import tpu as pltpu
from jax.experimental.pallas import tpu_sc as plsc

NN = 100000
EE = 1600000
GG = 64
F = 32
NC, NS, LN = 2, 16, 16
NW = NC * NS
NB = 32
CSP = 3200
NPAD = NB * CSP
EPW = EE // NW
CH = 2000
NCH = EPW // CH
EPAD = 1671168
EPW2 = EPAD // NW
S2CB = 512
S2NF = EPW2 // S2CB
S2C0 = 58
ESZ = EPAD
LSZ = EPW + NB * 64
BR1 = 6400
BR3 = 4096
EP4 = EPAD // 4
DW = 8

_SC_PARAMS = pltpu.CompilerParams(
    needs_layout_passes=False, use_tc_tiling_on_sc=False)


def _mesh():
    return plsc.VectorSubcoreMesh(
        core_axis_name="c", subcore_axis_name="s",
        num_cores=NC, num_subcores=NS)


def _wid():
    return lax.axis_index("s") * NC + lax.axis_index("c")


def _bkt(d):
    return ((d >> 7) * 5243) >> 17


def _lanes():
    return lax.iota(jnp.int32, LN)


def _prefix(cb, cs):
    lanes = _lanes()
    carry = jnp.int32(0)
    for k in range(NW * NB // LN):
        j = k * LN + lanes
        b = j >> 5
        wv = j & 31
        idx = wv * NB + b
        c = plsc.load_gather(cb, [idx])
        cp = (c + 63) & (-64)
        excl = plsc.cumsum(cp) - cp + carry
        plsc.store_scatter(cs, [idx], excl)
        carry = carry + jnp.sum(cp)
    return carry


def _pa_body(dst_hbm, counts_hbm, dbuf, hist, cbuf, sem):
    del sem
    w = _wid()
    lanes = _lanes()
    for b in range(NB):
        hist[pl.ds(b * LN, LN)] = jnp.zeros((LN,), jnp.int32)

    def chunk(c, car):
        pltpu.sync_copy(dst_hbm.at[pl.ds(pl.multiple_of(w * EPW + c * CH, 8), CH)], dbuf)

        def hb(v, car2):
            d = dbuf[pl.ds(v * LN, LN)]
            fi = _bkt(d) * LN + lanes
            plsc.store_scatter(hist, [fi], plsc.load_gather(hist, [fi]) + 1)
            return car2

        return lax.fori_loop(0, CH // LN, hb, car)

    lax.fori_loop(0, NCH, chunk, 0)
    for g in range(NB // LN):
        acc = jnp.zeros((LN,), jnp.int32)
        for l in range(LN):
            acc = acc + plsc.load_gather(hist, [(lanes + g * LN) * LN + l])
        cbuf[pl.ds(g * LN, LN)] = acc
    pltpu.sync_copy(cbuf, counts_hbm.at[pl.ds(pl.multiple_of(w * NB, 8), NB)])


def _run_pa(dst):
    kfn = functools.partial(
        pl.kernel, mesh=_mesh(), compiler_params=_SC_PARAMS,
        out_type=jax.ShapeDtypeStruct((NW * NB,), jnp.int32),
        scratch_types=[
            pltpu.VMEM((CH,), jnp.int32),
            pltpu.VMEM((NB * LN,), jnp.int32),
            pltpu.VMEM((NB,), jnp.int32),
            pltpu.SemaphoreType.DMA,
        ],
    )(_pa_body)
    return kfn(dst)


def _pc_body(src_hbm, dst_hbm, counts_hbm, srcp_hbm, dstp_hbm,
             dbuf, sbuf, cb, cs, fl, ll0, loc_s, loc_d, sem):
    w = _wid()
    lanes = _lanes()
    pltpu.sync_copy(counts_hbm, cb)
    _prefix(cb, cs)
    lcarry = jnp.int32(0)
    for g in range(NB // LN):
        c = cb[pl.ds(w * NB + g * LN, LN)]
        cp = (c + 63) & (-64)
        excl = plsc.cumsum(cp) - cp + lcarry
        fl[pl.ds(g * LN, LN)] = excl
        ll0[pl.ds(g * LN, LN)] = excl
        lcarry = lcarry + jnp.sum(cp)

    def chunk(c, car):
        base_e = pl.multiple_of(w * EPW + c * CH, 8)
        pltpu.sync_copy(dst_hbm.at[pl.ds(base_e, CH)], dbuf)
        pltpu.sync_copy(src_hbm.at[pl.ds(base_e, CH)], sbuf)

        def vec(v, car2):
            d = dbuf[pl.ds(v * LN, LN)]
            s = sbuf[pl.ds(v * LN, LN)]
            kd, pv = plsc.sort_key_val(d, lanes)
            sv = s.at[pv].get(mode="promise_in_bounds")
            b = _bkt(kd)
            bprev = b.at[jnp.maximum(lanes - 1, 0)].get(
                mode="promise_in_bounds")
            mnew = (b != bprev) | (lanes == 0)
            runstart = plsc.cummax(jnp.where(mnew, lanes, -1))
            rank = lanes - runstart
            pos = plsc.load_gather(fl, [b]) + rank
            plsc.store_scatter(loc_s, [pos], sv)
            plsc.store_scatter(loc_d, [pos], kd)
            bnext = b.at[jnp.minimum(lanes + 1, LN - 1)].get(
                mode="promise_in_bounds")
            mend = (b != bnext) | (lanes == LN - 1)
            plsc.store_scatter(fl, [b], pos + 1, mask=mend)
            return car2

        lax.fori_loop(0, CH // LN, vec, 0)
        return car

    lax.fori_loop(0, NCH, chunk, 0)

    for b in range(NB):
        g0 = (b // LN) * LN
        f = fl[pl.ds(g0, LN)][b % LN]
        l0 = ll0[pl.ds(g0, LN)][b % LN]
        cnt_b = f - l0
        cpb = (cnt_b + 63) & (-64)
        npad = cpb - cnt_b
        lastix = jnp.maximum(f - 1, 0)
        vs = plsc.load_gather(loc_s, [jnp.full((LN,), lastix, jnp.int32)])
        vd = plsc.load_gather(loc_d, [jnp.full((LN,), lastix, jnp.int32)])
        for t in range(4):
            tl = t * LN + lanes
            idxp = f + tl
            mpad = tl < npad
            plsc.store_scatter(loc_s, [idxp], vs, mask=mpad)
            plsc.store_scatter(loc_d, [idxp], vd, mask=mpad)
        gc0 = plsc.load_gather(
            cs, [jnp.full((LN,), w * NB + b, jnp.int32)])[0]
        n64 = cpb >> 6

        def is64(j, car, l0=l0, gc0=gc0):
            so = pl.multiple_of(l0 + j * 64, 8)
            do = pl.multiple_of(gc0 + j * 64, 8)
            pltpu.async_copy(loc_s.at[pl.ds(so, 64)],
                             srcp_hbm.at[pl.ds(do, 64)], sem)
            pltpu.async_copy(loc_d.at[pl.ds(so, 64)],
                             dstp_hbm.at[pl.ds(do, 64)], sem)

            @pl.when(j >= DW)
            def _drain(j=j, l0=l0, gc0=gc0):
                sod = pl.multiple_of(l0 + (j - DW) * 64, 8)
                dod = pl.multiple_of(gc0 + (j - DW) * 64, 8)
                pltpu.make_async_copy(
                    loc_s.at[pl.ds(sod, 64)],
                    srcp_hbm.at[pl.ds(dod, 64)], sem).wait()
                pltpu.make_async_copy(
                    loc_d.at[pl.ds(sod, 64)],
                    dstp_hbm.at[pl.ds(dod, 64)], sem).wait()
            return car

        lax.fori_loop(0, n64, is64, 0)

        def drain64(j, car, l0=l0, gc0=gc0):
            sod = pl.multiple_of(l0 + j * 64, 8)
            dod = pl.multiple_of(gc0 + j * 64, 8)
            pltpu.make_async_copy(
                loc_s.at[pl.ds(sod, 64)],
                srcp_hbm.at[pl.ds(dod, 64)], sem).wait()
            pltpu.make_async_copy(
                loc_d.at[pl.ds(sod, 64)],
                dstp_hbm.at[pl.ds(dod, 64)], sem).wait()
            return car

        lax.fori_loop(jnp.maximum(n64 - DW, 0), n64, drain64, 0)


def _run_pc(src, dst, counts):
    kfn = functools.partial(
        pl.kernel, mesh=_mesh(), compiler_params=_SC_PARAMS,
        out_type=[jax.ShapeDtypeStruct((ESZ,), jnp.int32),
                  jax.ShapeDtypeStruct((ESZ,), jnp.int32)],
        scratch_types=[
            pltpu.VMEM((CH,), jnp.int32),
            pltpu.VMEM((CH,), jnp.int32),
            pltpu.VMEM((NW * NB,), jnp.int32),
            pltpu.VMEM((NW * NB,), jnp.int32),
            pltpu.VMEM((NB,), jnp.int32),
            pltpu.VMEM((NB,), jnp.int32),
            pltpu.VMEM((LSZ,), jnp.int32),
            pltpu.VMEM((LSZ,), jnp.int32),
            pltpu.SemaphoreType.DMA,
        ],
    )(_pc_body)
    return kfn(src, dst, counts)


def _s2_body(srcp_hbm, dstp_hbm, u_hbm, v_hbm, z_hbm,
             sidx, didx, ubuf, vbuf, zbuf, isem, gsem_a, gsem_b):
    core = lax.axis_index("c")
    sub = lax.axis_index("s")
    c_slow = S2C0 // 2
    c_fast = (2 * S2NF - S2C0) // 2
    nc2 = jnp.where(core == 0, c_slow, c_fast)
    base = pl.multiple_of(
        sub * (2 * EPW2) + jnp.where(core == 0, 0, S2C0 * S2CB), 8)
    gsems = (gsem_a, gsem_b)

    def prefetch(eoff, p):
        for j in range(4):
            off_j = pl.multiple_of(eoff + j * 128, 8)
            pltpu.async_copy(srcp_hbm.at[pl.ds(off_j, 128)],
                             sidx.at[p * 4 + j], isem)
            pltpu.async_copy(dstp_hbm.at[pl.ds(off_j, 128)],
                             didx.at[p * 4 + j], isem)
        for j in range(4):
            off_j = pl.multiple_of(eoff + j * 128, 8)
            pltpu.make_async_copy(srcp_hbm.at[pl.ds(off_j, 128)],
                                  sidx.at[p * 4 + j], isem).wait()
            pltpu.make_async_copy(dstp_hbm.at[pl.ds(off_j, 128)],
                                  didx.at[p * 4 + j], isem).wait()
        for j in range(4):
            for k in range(8):
                iv = sidx[p * 4 + j, pl.ds(k * LN, LN)]
                sidx[p * 4 + j, pl.ds(k * LN, LN)] = jnp.minimum(
                    jnp.maximum(iv, 0), NPAD - 1)
                iv2 = didx[p * 4 + j, pl.ds(k * LN, LN)]
                didx[p * 4 + j, pl.ds(k * LN, LN)] = jnp.minimum(
                    jnp.maximum(iv2, 0), NPAD - 1)
        for j in range(4):
            pltpu.async_copy(u_hbm.at[sidx.at[p * 4 + j]],
                             ubuf.at[pl.ds(p * 512 + j * 128, 128)], gsems[p])
            pltpu.async_copy(v_hbm.at[didx.at[p * 4 + j]],
                             vbuf.at[pl.ds(p * 512 + j * 128, 128)], gsems[p])

    def consume(eoff, p):
        for j in range(4):
            pltpu.make_async_copy(
                u_hbm.at[sidx.at[p * 4 + j]],
                ubuf.at[pl.ds(p * 512 + j * 128, 128)], gsems[p]).wait()
            pltpu.make_async_copy(
                v_hbm.at[didx.at[p * 4 + j]],
                vbuf.at[pl.ds(p * 512 + j * 128, 128)], gsems[p]).wait()

        def cz(i2, car):
            for q in range(4):
                for h in range(2):
                    e = p * 512 + i2 * 4 + q
                    zbuf[i2, pl.ds(q * 32 + h * LN, LN)] = jnp.maximum(
                        ubuf[e, pl.ds(h * LN, LN)] + vbuf[e, pl.ds(h * LN, LN)],
                        0.0)
            return car

        lax.fori_loop(0, 128, cz, 0)
        pltpu.sync_copy(
            zbuf, z_hbm.at[pl.ds(pl.multiple_of(eoff >> 2, 8), 128)])

    prefetch(base, 0)

    def loop(c2, car):
        e0 = base + (2 * c2) * S2CB
        e1 = base + (2 * c2 + 1) * S2CB
        prefetch(e1, 1)
        consume(e0, 0)

        @pl.when(c2 < nc2 - 1)
        def _pf():
            prefetch(e1 + S2CB, 0)
        consume(e1, 1)
        return car

    lax.fori_loop(0, nc2, loop, 0)


def _run_s2(srcp, dstp, u, v):
    kfn = functools.partial(
        pl.kernel, mesh=_mesh(), compiler_params=_SC_PARAMS,
        out_type=jax.ShapeDtypeStruct((EP4, 128), jnp.float32),
        scratch_types=[
            pltpu.VMEM((8, 128), jnp.int32),
            pltpu.VMEM((8, 128), jnp.int32),
            pltpu.VMEM((1024, F), jnp.float32),
            pltpu.VMEM((1024, F), jnp.float32),
            pltpu.VMEM((128, 128), jnp.float32),
            pltpu.SemaphoreType.DMA,
            pltpu.SemaphoreType.DMA,
            pltpu.SemaphoreType.DMA,
        ],
    )(_s2_body)
    return kfn(srcp, dstp, u, v)


def _s4_common(m_hbm, dstp_hbm, counts_hbm, cb, cs, mbuf, dbuf4, tbl, sem):
    w = _wid()
    pltpu.sync_copy(counts_hbm, cb)
    total = _prefix(cb, cs)
    s = plsc.load_gather(cs, [jnp.full((LN,), w, jnp.int32)])[0]
    e_next = plsc.load_gather(
        cs, [jnp.full((LN,), jnp.minimum(w + 1, NB - 1), jnp.int32)])[0]
    e = jnp.where(w == NB - 1, total, e_next)
    nodebase = w * CSP

    def zt(i, car):
        for h in range(2):
            tbl[i, pl.ds(h * LN, LN)] = jnp.zeros((LN,), jnp.float32)
        return car

    lax.fori_loop(0, CSP, zt, 0)
    dbuf4[pl.ds(0, LN)] = jnp.full((LN,), nodebase, jnp.int32)
    for r in range(4):
        for kk in range(8):
            mbuf[r, pl.ds(kk * LN, LN)] = jnp.zeros((LN,), jnp.float32)

    def apply_grp(g, p):
        dv = dbuf4[pl.ds(p * 256 + g * LN, LN)]
        dv = jnp.minimum(jnp.maximum(dv - nodebase, 0), CSP - 1)
        for j in range(LN):
            r = dv[j]
            mrow = p * 64 + g * 4 + (j >> 2)
            for h in range(2):
                mcol = (j & 3) * 32 + h * LN
                tbl[r, pl.ds(h * LN, LN)] = jnp.maximum(
                    tbl[r, pl.ds(h * LN, LN)], mbuf[mrow, pl.ds(mcol, LN)])

    cnt = e - s
    nfull = cnt >> 8

    def fire(c, p):
        off = pl.multiple_of(s + c * 256, 8)
        moff = pl.multiple_of((s + c * 256) >> 2, 8)
        pltpu.async_copy(m_hbm.at[pl.ds(moff, 64)],
                         mbuf.at[pl.ds(p * 64, 64)], sem)
        pltpu.async_copy(dstp_hbm.at[pl.ds(off, 256)],
                         dbuf4.at[pl.ds(p * 256, 256)], sem)

    def drain(c, p):
        off = pl.multiple_of(s + c * 256, 8)
        moff = pl.multiple_of((s + c * 256) >> 2, 8)
        pltpu.make_async_copy(m_hbm.at[pl.ds(moff, 64)],
                              mbuf.at[pl.ds(p * 64, 64)], sem).wait()
        pltpu.make_async_copy(dstp_hbm.at[pl.ds(off, 256)],
                              dbuf4.at[pl.ds(p * 256, 256)], sem).wait()

    @pl.when(nfull > 0)
    def _p0():
        fire(0, 0)

    def chunk(c, car):
        p = c & 1
        drain(c, p)

        @pl.when(c + 1 < nfull)
        def _pf(c=c, p=p):
            fire(c + 1, 1 - p)

        def grp(g, car2):
            apply_grp(g, p)
            return car2

        lax.fori_loop(0, 256 // LN, grp, 0)
        return car

    lax.fori_loop(0, nfull, chunk, 0)
    t0 = s + (nfull << 8)
    ng8 = (e - t0) >> 3

    def g8(j, car):
        off = pl.multiple_of(t0 + j * 8, 8)
        pltpu.sync_copy(m_hbm.at[pl.ds((t0 + j * 8) >> 2, 2)],
                        mbuf.at[pl.ds(0, 2)])
        pltpu.sync_copy(dstp_hbm.at[pl.ds(off, 8)], dbuf4.at[pl.ds(0, 8)])
        apply_grp(0, 0)
        return car

    lax.fori_loop(0, ng8, g8, 0)
    return w, nodebase


def _s4h_body(m_hbm, dstp_hbm, counts_hbm, h_hbm,
              cb, cs, mbuf, dbuf4, tbl, sem):
    w, nodebase = _s4_common(
        m_hbm, dstp_hbm, counts_hbm, cb, cs, mbuf, dbuf4, tbl, sem)
    pltpu.sync_copy(tbl, h_hbm.at[pl.ds(pl.multiple_of(nodebase, 8), CSP)])


def _s4g_body(m_hbm, dstp_hbm, counts_hbm, batch_hbm, gpart_hbm,
              cb, cs, mbuf, dbuf4, tbl, bbuf, gtbl, sem):
    w, nodebase = _s4_common(
        m_hbm, dstp_hbm, counts_hbm, cb, cs, mbuf, dbuf4, tbl, sem)
    pltpu.sync_copy(batch_hbm.at[pl.ds(pl.multiple_of(nodebase, 8), CSP)], bbuf)
    for r in range(GG):
        for h in range(2):
            gtbl[r, pl.ds(h * LN, LN)] = jnp.zeros((LN,), jnp.float32)

    def pool(rg, car):
        bv = bbuf[pl.ds(rg * LN, LN)]
        bv = jnp.minimum(jnp.maximum(bv, 0), GG - 1)
        for j in range(LN):
            gi = bv[j]
            nr = rg * LN + j
            for h in range(2):
                gtbl[gi, pl.ds(h * LN, LN)] = jnp.maximum(
                    gtbl[gi, pl.ds(h * LN, LN)], tbl[nr, pl.ds(h * LN, LN)])
        return car

    lax.fori_loop(0, CSP // LN, pool, 0)
    pltpu.sync_copy(gtbl, gpart_hbm.at[w])


def _s4_scratch():
    return [
        pltpu.VMEM((NW * NB,), jnp.int32),
        pltpu.VMEM((NW * NB,), jnp.int32),
        pltpu.VMEM((128, 128), jnp.float32),
        pltpu.VMEM((512,), jnp.int32),
        pltpu.VMEM((CSP, F), jnp.float32),
    ]


def _run_s4h(m, dstp, counts):
    kfn = functools.partial(
        pl.kernel, mesh=_mesh(), compiler_params=_SC_PARAMS,
        out_type=jax.ShapeDtypeStruct((NPAD, F), jnp.float32),
        scratch_types=_s4_scratch() + [pltpu.SemaphoreType.DMA],
    )(_s4h_body)
    return kfn(m, dstp, counts)


def _run_s4g(m, dstp, counts, batch_pad):
    kfn = functools.partial(
        pl.kernel, mesh=_mesh(), compiler_params=_SC_PARAMS,
        out_type=jax.ShapeDtypeStruct((NW, GG, F), jnp.float32),
        scratch_types=_s4_scratch() + [
            pltpu.VMEM((CSP,), jnp.int32),
            pltpu.VMEM((GG, F), jnp.float32),
            pltpu.SemaphoreType.DMA,
        ],
    )(_s4g_body)
    return kfn(m, dstp, counts, batch_pad)


def _s1a_body(pos_ref, w_ref, b_ref, u_ref, v_ref):
    wfull = w_ref[...]
    wh = wfull[0:3] + wfull[3:6]
    wp = wfull[3:6]
    p = pos_ref[...]
    u_ref[...] = jnp.dot(p, wh, preferred_element_type=jnp.float32) + b_ref[...]
    v_ref[...] = -jnp.dot(p, wp, preferred_element_type=jnp.float32)


def _run_s1a(pos_pad, w1, b1):
    grid = NPAD // BR1
    return pl.pallas_call(
        _s1a_body,
        grid=(grid,),
        in_specs=[
            pl.BlockSpec((BR1, 3), lambda i: (i, 0)),
            pl.BlockSpec((6, F), lambda i: (0, 0)),
            pl.BlockSpec((1, F), lambda i: (0, 0)),
        ],
        out_specs=[
            pl.BlockSpec((BR1, F), lambda i: (i, 0)),
            pl.BlockSpec((BR1, F), lambda i: (i, 0)),
        ],
        out_shape=[jax.ShapeDtypeStruct((NPAD, F), jnp.float32),
                   jax.ShapeDtypeStruct((NPAD, F), jnp.float32)],
    )(pos_pad, w1, b1)


def _s1b_body(h_ref, pos_ref, w_ref, b_ref, u_ref, v_ref):
    wfull = w_ref[...]
    wh = wfull[0:F]
    wp = wfull[F:F + 3]
    p = pos_ref[...]
    pv = jnp.dot(p, wp, preferred_element_type=jnp.float32)
    u_ref[...] = (jnp.dot(h_ref[...], wh, preferred_element_type=jnp.float32)
                  + pv + b_ref[...])
    v_ref[...] = -pv


def _run_s1b(h1, pos_pad, w1, b1):
    grid = NPAD // BR1
    return pl.pallas_call(
        _s1b_body,
        grid=(grid,),
        in_specs=[
            pl.BlockSpec((BR1, F), lambda i: (i, 0)),
            pl.BlockSpec((BR1, 3), lambda i: (i, 0)),
            pl.BlockSpec((F + 3, F), lambda i: (0, 0)),
            pl.BlockSpec((1, F), lambda i: (0, 0)),
        ],
        out_specs=[
            pl.BlockSpec((BR1, F), lambda i: (i, 0)),
            pl.BlockSpec((BR1, F), lambda i: (i, 0)),
        ],
        out_shape=[jax.ShapeDtypeStruct((NPAD, F), jnp.float32),
                   jax.ShapeDtypeStruct((NPAD, F), jnp.float32)],
    )(h1, pos_pad, w1, b1)


def _s3_body(z_ref, w_ref, b_ref, m_ref):
    m_ref[...] = (jnp.dot(z_ref[...], w_ref[...],
                          preferred_element_type=jnp.float32) + b_ref[...])


def _run_s3(z, w2bd, b2t):
    grid = EP4 // (BR3 // 4)
    return pl.pallas_call(
        _s3_body,
        grid=(grid,),
        in_specs=[
            pl.BlockSpec((BR3 // 4, 128), lambda i: (i, 0)),
            pl.BlockSpec((128, 128), lambda i: (0, 0)),
            pl.BlockSpec((1, 128), lambda i: (0, 0)),
        ],
        out_specs=pl.BlockSpec((BR3 // 4, 128), lambda i: (i, 0)),
        out_shape=jax.ShapeDtypeStruct((EP4, 128), jnp.float32),
    )(z, w2bd, b2t)


def _fin_body(gp_ref, wc_ref, bc_ref, out_ref):
    g = jnp.max(gp_ref[...], axis=0)
    out_ref[...] = (jnp.dot(g, wc_ref[...],
                            preferred_element_type=jnp.float32) + bc_ref[...])


def _run_fin(gpart, wc, bc):
    return pl.pallas_call(
        _fin_body,
        out_shape=jax.ShapeDtypeStruct((GG, wc.shape[1]), jnp.float32),
    )(gpart, wc, bc)


def kernel(pos, edge_index, batch, W1a, b1a, W2a, b2a, W1b, b1b, W2b, b2b,
           Wc, bc):
    src = edge_index[0]
    dst = edge_index[1]
    pos_pad = jnp.pad(pos, ((0, NPAD - NN), (0, 0)))
    batch_pad = jnp.pad(batch, (0, NPAD - NN))

    zf = jnp.zeros((F, F), jnp.float32)
    w2bd_a = jnp.block([[W2a, zf, zf, zf], [zf, W2a, zf, zf],
                        [zf, zf, W2a, zf], [zf, zf, zf, W2a]])
    w2bd_b = jnp.block([[W2b, zf, zf, zf], [zf, W2b, zf, zf],
                        [zf, zf, W2b, zf], [zf, zf, zf, W2b]])
    b2t_a = jnp.tile(b2a, 4).reshape(1, 128)
    b2t_b = jnp.tile(b2b, 4).reshape(1, 128)

    counts = _run_pa(dst)
    src_p, dst_p = _run_pc(src, dst, counts)

    u1, v1 = _run_s1a(pos_pad, W1a, b1a.reshape(1, F))
    z1 = _run_s2(src_p, dst_p, u1, v1)
    m1 = _run_s3(z1, w2bd_a, b2t_a)
    h1 = _run_s4h(m1, dst_p, counts)

    u2, v2 = _run_s1b(h1, pos_pad, W1b, b1b.reshape(1, F))
    z2 = _run_s2(src_p, dst_p, u2, v2)
    m2 = _run_s3(z2, w2bd_b, b2t_b)
    gpart = _run_s4g(m2, dst_p, counts, batch_pad)

    return _run_fin(gpart, Wc, bc.reshape(1, -1))


# speedup vs baseline: 5.0486x; 1.0130x over previous
import functools

import jax
import jax.numpy as jnp
from jax import lax
from jax.experimental import pallas as pl
from jax.experimental.pallas import tpu as pltpu
from jax.experimental.pallas import tpu_sc as plsc

NN = 100000
EE = 1600000
GG = 64
F = 32
NC, NS, LN = 2, 16, 16
NW = NC * NS
NB = 32
CSP = 3200
NPAD = NB * CSP
EPW = EE // NW
CH = 2000
NCH = EPW // CH
EPAD = 1671168
EPW2 = EPAD // NW
S2CB = 512
S2NF = EPW2 // S2CB
S2C0 = 58
ESZ = EPAD
LSZ = EPW + NB * 64
BR1 = 6400
BR3 = 4096
EP4 = EPAD // 4
DW = 8

_SC_PARAMS = pltpu.CompilerParams(
    needs_layout_passes=False, use_tc_tiling_on_sc=False)


def _mesh():
    return plsc.VectorSubcoreMesh(
        core_axis_name="c", subcore_axis_name="s",
        num_cores=NC, num_subcores=NS)


def _wid():
    return lax.axis_index("s") * NC + lax.axis_index("c")


def _bkt(d):
    return ((d >> 7) * 5243) >> 17


def _lanes():
    return lax.iota(jnp.int32, LN)


def _prefix(cb, cs):
    lanes = _lanes()
    carry = jnp.int32(0)
    for k in range(NW * NB // LN):
        j = k * LN + lanes
        b = j >> 5
        wv = j & 31
        idx = wv * NB + b
        c = plsc.load_gather(cb, [idx])
        cp = (c + 63) & (-64)
        excl = plsc.cumsum(cp) - cp + carry
        plsc.store_scatter(cs, [idx], excl)
        carry = carry + jnp.sum(cp)
    return carry


def _pa_body(dst_hbm, counts_hbm, dbuf, hist, cbuf, sem):
    del sem
    w = _wid()
    lanes = _lanes()
    for b in range(NB):
        hist[pl.ds(b * LN, LN)] = jnp.zeros((LN,), jnp.int32)

    def chunk(c, car):
        pltpu.sync_copy(dst_hbm.at[pl.ds(pl.multiple_of(w * EPW + c * CH, 8), CH)], dbuf)

        def hb(v, car2):
            d = dbuf[pl.ds(v * LN, LN)]
            fi = _bkt(d) * LN + lanes
            plsc.store_scatter(hist, [fi], plsc.load_gather(hist, [fi]) + 1)
            return car2

        return lax.fori_loop(0, CH // LN, hb, car)

    lax.fori_loop(0, NCH, chunk, 0)
    for g in range(NB // LN):
        acc = jnp.zeros((LN,), jnp.int32)
        for l in range(LN):
            acc = acc + plsc.load_gather(hist, [(lanes + g * LN) * LN + l])
        cbuf[pl.ds(g * LN, LN)] = acc
    pltpu.sync_copy(cbuf, counts_hbm.at[pl.ds(pl.multiple_of(w * NB, 8), NB)])


def _run_pa(dst):
    kfn = functools.partial(
        pl.kernel, mesh=_mesh(), compiler_params=_SC_PARAMS,
        out_type=jax.ShapeDtypeStruct((NW * NB,), jnp.int32),
        scratch_types=[
            pltpu.VMEM((CH,), jnp.int32),
            pltpu.VMEM((NB * LN,), jnp.int32),
            pltpu.VMEM((NB,), jnp.int32),
            pltpu.SemaphoreType.DMA,
        ],
    )(_pa_body)
    return kfn(dst)


def _pc_body(src_hbm, dst_hbm, counts_hbm, srcp_hbm, dstp_hbm,
             dbuf, sbuf, cb, cs, fl, ll0, loc_s, loc_d, sem):
    w = _wid()
    lanes = _lanes()
    pltpu.sync_copy(counts_hbm, cb)
    _prefix(cb, cs)
    lcarry = jnp.int32(0)
    for g in range(NB // LN):
        c = cb[pl.ds(w * NB + g * LN, LN)]
        cp = (c + 63) & (-64)
        excl = plsc.cumsum(cp) - cp + lcarry
        fl[pl.ds(g * LN, LN)] = excl
        ll0[pl.ds(g * LN, LN)] = excl
        lcarry = lcarry + jnp.sum(cp)

    def chunk(c, car):
        base_e = pl.multiple_of(w * EPW + c * CH, 8)
        pltpu.sync_copy(dst_hbm.at[pl.ds(base_e, CH)], dbuf)
        pltpu.sync_copy(src_hbm.at[pl.ds(base_e, CH)], sbuf)

        def vec(v, car2):
            d = dbuf[pl.ds(v * LN, LN)]
            s = sbuf[pl.ds(v * LN, LN)]
            kd, pv = plsc.sort_key_val(d, lanes)
            sv = s.at[pv].get(mode="promise_in_bounds")
            b = _bkt(kd)
            bprev = b.at[jnp.maximum(lanes - 1, 0)].get(
                mode="promise_in_bounds")
            mnew = (b != bprev) | (lanes == 0)
            runstart = plsc.cummax(jnp.where(mnew, lanes, -1))
            rank = lanes - runstart
            pos = plsc.load_gather(fl, [b]) + rank
            plsc.store_scatter(loc_s, [pos], sv)
            plsc.store_scatter(loc_d, [pos], kd)
            bnext = b.at[jnp.minimum(lanes + 1, LN - 1)].get(
                mode="promise_in_bounds")
            mend = (b != bnext) | (lanes == LN - 1)
            plsc.store_scatter(fl, [b], pos + 1, mask=mend)
            return car2

        lax.fori_loop(0, CH // LN, vec, 0)
        return car

    lax.fori_loop(0, NCH, chunk, 0)

    for b in range(NB):
        g0 = (b // LN) * LN
        f = fl[pl.ds(g0, LN)][b % LN]
        l0 = ll0[pl.ds(g0, LN)][b % LN]
        cnt_b = f - l0
        cpb = (cnt_b + 63) & (-64)
        npad = cpb - cnt_b
        lastix = jnp.maximum(f - 1, 0)
        vs = plsc.load_gather(loc_s, [jnp.full((LN,), lastix, jnp.int32)])
        vd = plsc.load_gather(loc_d, [jnp.full((LN,), lastix, jnp.int32)])
        for t in range(4):
            tl = t * LN + lanes
            idxp = f + tl
            mpad = tl < npad
            plsc.store_scatter(loc_s, [idxp], vs, mask=mpad)
            plsc.store_scatter(loc_d, [idxp], vd, mask=mpad)
        gc0 = plsc.load_gather(
            cs, [jnp.full((LN,), w * NB + b, jnp.int32)])[0]
        n64 = cpb >> 6

        def is64(j, car, l0=l0, gc0=gc0):
            so = pl.multiple_of(l0 + j * 64, 8)
            do = pl.multiple_of(gc0 + j * 64, 8)
            pltpu.async_copy(loc_s.at[pl.ds(so, 64)],
                             srcp_hbm.at[pl.ds(do, 64)], sem)
            pltpu.async_copy(loc_d.at[pl.ds(so, 64)],
                             dstp_hbm.at[pl.ds(do, 64)], sem)

            @pl.when(j >= DW)
            def _drain(j=j, l0=l0, gc0=gc0):
                sod = pl.multiple_of(l0 + (j - DW) * 64, 8)
                dod = pl.multiple_of(gc0 + (j - DW) * 64, 8)
                pltpu.make_async_copy(
                    loc_s.at[pl.ds(sod, 64)],
                    srcp_hbm.at[pl.ds(dod, 64)], sem).wait()
                pltpu.make_async_copy(
                    loc_d.at[pl.ds(sod, 64)],
                    dstp_hbm.at[pl.ds(dod, 64)], sem).wait()
            return car

        lax.fori_loop(0, n64, is64, 0)

        def drain64(j, car, l0=l0, gc0=gc0):
            sod = pl.multiple_of(l0 + j * 64, 8)
            dod = pl.multiple_of(gc0 + j * 64, 8)
            pltpu.make_async_copy(
                loc_s.at[pl.ds(sod, 64)],
                srcp_hbm.at[pl.ds(dod, 64)], sem).wait()
            pltpu.make_async_copy(
                loc_d.at[pl.ds(sod, 64)],
                dstp_hbm.at[pl.ds(dod, 64)], sem).wait()
            return car

        lax.fori_loop(jnp.maximum(n64 - DW, 0), n64, drain64, 0)


def _run_pc(src, dst, counts):
    kfn = functools.partial(
        pl.kernel, mesh=_mesh(), compiler_params=_SC_PARAMS,
        out_type=[jax.ShapeDtypeStruct((ESZ,), jnp.int32),
                  jax.ShapeDtypeStruct((ESZ,), jnp.int32)],
        scratch_types=[
            pltpu.VMEM((CH,), jnp.int32),
            pltpu.VMEM((CH,), jnp.int32),
            pltpu.VMEM((NW * NB,), jnp.int32),
            pltpu.VMEM((NW * NB,), jnp.int32),
            pltpu.VMEM((NB,), jnp.int32),
            pltpu.VMEM((NB,), jnp.int32),
            pltpu.VMEM((LSZ,), jnp.int32),
            pltpu.VMEM((LSZ,), jnp.int32),
            pltpu.SemaphoreType.DMA,
        ],
    )(_pc_body)
    return kfn(src, dst, counts)


def _s2_body(srcp_hbm, dstp_hbm, u_hbm, v_hbm, z_hbm,
             sidx, didx, ubuf, vbuf, zbuf, isem, gsem_a, gsem_b):
    core = lax.axis_index("c")
    sub = lax.axis_index("s")
    c_slow = S2C0 // 2
    c_fast = (2 * S2NF - S2C0) // 2
    nc2 = jnp.where(core == 1, c_slow, c_fast)
    base = pl.multiple_of(
        sub * (2 * EPW2) + jnp.where(core == 1, 0, S2C0 * S2CB), 8)
    gsems = (gsem_a, gsem_b)

    def prefetch(eoff, p):
        for j in range(4):
            off_j = pl.multiple_of(eoff + j * 128, 8)
            pltpu.async_copy(srcp_hbm.at[pl.ds(off_j, 128)],
                             sidx.at[p * 4 + j], isem)
            pltpu.async_copy(dstp_hbm.at[pl.ds(off_j, 128)],
                             didx.at[p * 4 + j], isem)
        for j in range(4):
            off_j = pl.multiple_of(eoff + j * 128, 8)
            pltpu.make_async_copy(srcp_hbm.at[pl.ds(off_j, 128)],
                                  sidx.at[p * 4 + j], isem).wait()
            pltpu.make_async_copy(dstp_hbm.at[pl.ds(off_j, 128)],
                                  didx.at[p * 4 + j], isem).wait()
        for j in range(4):
            for k in range(8):
                iv = sidx[p * 4 + j, pl.ds(k * LN, LN)]
                sidx[p * 4 + j, pl.ds(k * LN, LN)] = jnp.minimum(
                    jnp.maximum(iv, 0), NPAD - 1)
                iv2 = didx[p * 4 + j, pl.ds(k * LN, LN)]
                didx[p * 4 + j, pl.ds(k * LN, LN)] = jnp.minimum(
                    jnp.maximum(iv2, 0), NPAD - 1)
        for j in range(4):
            pltpu.async_copy(u_hbm.at[sidx.at[p * 4 + j]],
                             ubuf.at[pl.ds(p * 512 + j * 128, 128)], gsems[p])
            pltpu.async_copy(v_hbm.at[didx.at[p * 4 + j]],
                             vbuf.at[pl.ds(p * 512 + j * 128, 128)], gsems[p])

    def consume(eoff, p):
        for j in range(4):
            pltpu.make_async_copy(
                u_hbm.at[sidx.at[p * 4 + j]],
                ubuf.at[pl.ds(p * 512 + j * 128, 128)], gsems[p]).wait()
            pltpu.make_async_copy(
                v_hbm.at[didx.at[p * 4 + j]],
                vbuf.at[pl.ds(p * 512 + j * 128, 128)], gsems[p]).wait()

        def cz(i2, car):
            for q in range(4):
                for h in range(2):
                    e = p * 512 + i2 * 4 + q
                    zbuf[i2, pl.ds(q * 32 + h * LN, LN)] = jnp.maximum(
                        ubuf[e, pl.ds(h * LN, LN)] + vbuf[e, pl.ds(h * LN, LN)],
                        0.0)
            return car

        lax.fori_loop(0, 128, cz, 0)
        pltpu.sync_copy(
            zbuf, z_hbm.at[pl.ds(pl.multiple_of(eoff >> 2, 8), 128)])

    prefetch(base, 0)

    def loop(c2, car):
        e0 = base + (2 * c2) * S2CB
        e1 = base + (2 * c2 + 1) * S2CB
        prefetch(e1, 1)
        consume(e0, 0)

        @pl.when(c2 < nc2 - 1)
        def _pf():
            prefetch(e1 + S2CB, 0)
        consume(e1, 1)
        return car

    lax.fori_loop(0, nc2, loop, 0)


def _run_s2(srcp, dstp, u, v):
    kfn = functools.partial(
        pl.kernel, mesh=_mesh(), compiler_params=_SC_PARAMS,
        out_type=jax.ShapeDtypeStruct((EP4, 128), jnp.float32),
        scratch_types=[
            pltpu.VMEM((8, 128), jnp.int32),
            pltpu.VMEM((8, 128), jnp.int32),
            pltpu.VMEM((1024, F), jnp.float32),
            pltpu.VMEM((1024, F), jnp.float32),
            pltpu.VMEM((128, 128), jnp.float32),
            pltpu.SemaphoreType.DMA,
            pltpu.SemaphoreType.DMA,
            pltpu.SemaphoreType.DMA,
        ],
    )(_s2_body)
    return kfn(srcp, dstp, u, v)


def _s4_common(m_hbm, dstp_hbm, counts_hbm, cb, cs, mbuf, dbuf4, tbl, sem):
    w = _wid()
    pltpu.sync_copy(counts_hbm, cb)
    total = _prefix(cb, cs)
    s = plsc.load_gather(cs, [jnp.full((LN,), w, jnp.int32)])[0]
    e_next = plsc.load_gather(
        cs, [jnp.full((LN,), jnp.minimum(w + 1, NB - 1), jnp.int32)])[0]
    e = jnp.where(w == NB - 1, total, e_next)
    nodebase = w * CSP

    def zt(i, car):
        for h in range(2):
            tbl[i, pl.ds(h * LN, LN)] = jnp.zeros((LN,), jnp.float32)
        return car

    lax.fori_loop(0, CSP, zt, 0)
    dbuf4[pl.ds(0, LN)] = jnp.full((LN,), nodebase, jnp.int32)
    for r in range(4):
        for kk in range(8):
            mbuf[r, pl.ds(kk * LN, LN)] = jnp.zeros((LN,), jnp.float32)

    def apply_grp(g, p):
        dv = dbuf4[pl.ds(p * 256 + g * LN, LN)]
        dv = jnp.minimum(jnp.maximum(dv - nodebase, 0), CSP - 1)
        for j in range(LN):
            r = dv[j]
            mrow = p * 64 + g * 4 + (j >> 2)
            for h in range(2):
                mcol = (j & 3) * 32 + h * LN
                tbl[r, pl.ds(h * LN, LN)] = jnp.maximum(
                    tbl[r, pl.ds(h * LN, LN)], mbuf[mrow, pl.ds(mcol, LN)])

    cnt = e - s
    nfull = cnt >> 8

    def fire(c, p):
        off = pl.multiple_of(s + c * 256, 8)
        moff = pl.multiple_of((s + c * 256) >> 2, 8)
        pltpu.async_copy(m_hbm.at[pl.ds(moff, 64)],
                         mbuf.at[pl.ds(p * 64, 64)], sem)
        pltpu.async_copy(dstp_hbm.at[pl.ds(off, 256)],
                         dbuf4.at[pl.ds(p * 256, 256)], sem)

    def drain(c, p):
        off = pl.multiple_of(s + c * 256, 8)
        moff = pl.multiple_of((s + c * 256) >> 2, 8)
        pltpu.make_async_copy(m_hbm.at[pl.ds(moff, 64)],
                              mbuf.at[pl.ds(p * 64, 64)], sem).wait()
        pltpu.make_async_copy(dstp_hbm.at[pl.ds(off, 256)],
                              dbuf4.at[pl.ds(p * 256, 256)], sem).wait()

    @pl.when(nfull > 0)
    def _p0():
        fire(0, 0)

    def chunk(c, car):
        p = c & 1
        drain(c, p)

        @pl.when(c + 1 < nfull)
        def _pf(c=c, p=p):
            fire(c + 1, 1 - p)

        def grp(g, car2):
            apply_grp(g, p)
            return car2

        lax.fori_loop(0, 256 // LN, grp, 0)
        return car

    lax.fori_loop(0, nfull, chunk, 0)
    t0 = s + (nfull << 8)
    ng8 = (e - t0) >> 3

    def g8(j, car):
        off = pl.multiple_of(t0 + j * 8, 8)
        pltpu.sync_copy(m_hbm.at[pl.ds((t0 + j * 8) >> 2, 2)],
                        mbuf.at[pl.ds(0, 2)])
        pltpu.sync_copy(dstp_hbm.at[pl.ds(off, 8)], dbuf4.at[pl.ds(0, 8)])
        apply_grp(0, 0)
        return car

    lax.fori_loop(0, ng8, g8, 0)
    return w, nodebase


def _s4h_body(m_hbm, dstp_hbm, counts_hbm, h_hbm,
              cb, cs, mbuf, dbuf4, tbl, sem):
    w, nodebase = _s4_common(
        m_hbm, dstp_hbm, counts_hbm, cb, cs, mbuf, dbuf4, tbl, sem)
    pltpu.sync_copy(tbl, h_hbm.at[pl.ds(pl.multiple_of(nodebase, 8), CSP)])


def _s4g_body(m_hbm, dstp_hbm, counts_hbm, batch_hbm, gpart_hbm,
              cb, cs, mbuf, dbuf4, tbl, bbuf, gtbl, sem):
    w, nodebase = _s4_common(
        m_hbm, dstp_hbm, counts_hbm, cb, cs, mbuf, dbuf4, tbl, sem)
    pltpu.sync_copy(batch_hbm.at[pl.ds(pl.multiple_of(nodebase, 8), CSP)], bbuf)
    for r in range(GG):
        for h in range(2):
            gtbl[r, pl.ds(h * LN, LN)] = jnp.zeros((LN,), jnp.float32)

    def pool(rg, car):
        bv = bbuf[pl.ds(rg * LN, LN)]
        bv = jnp.minimum(jnp.maximum(bv, 0), GG - 1)
        for j in range(LN):
            gi = bv[j]
            nr = rg * LN + j
            for h in range(2):
                gtbl[gi, pl.ds(h * LN, LN)] = jnp.maximum(
                    gtbl[gi, pl.ds(h * LN, LN)], tbl[nr, pl.ds(h * LN, LN)])
        return car

    lax.fori_loop(0, CSP // LN, pool, 0)
    pltpu.sync_copy(gtbl, gpart_hbm.at[w])


def _s4_scratch():
    return [
        pltpu.VMEM((NW * NB,), jnp.int32),
        pltpu.VMEM((NW * NB,), jnp.int32),
        pltpu.VMEM((128, 128), jnp.float32),
        pltpu.VMEM((512,), jnp.int32),
        pltpu.VMEM((CSP, F), jnp.float32),
    ]


def _run_s4h(m, dstp, counts):
    kfn = functools.partial(
        pl.kernel, mesh=_mesh(), compiler_params=_SC_PARAMS,
        out_type=jax.ShapeDtypeStruct((NPAD, F), jnp.float32),
        scratch_types=_s4_scratch() + [pltpu.SemaphoreType.DMA],
    )(_s4h_body)
    return kfn(m, dstp, counts)


def _run_s4g(m, dstp, counts, batch_pad):
    kfn = functools.partial(
        pl.kernel, mesh=_mesh(), compiler_params=_SC_PARAMS,
        out_type=jax.ShapeDtypeStruct((NW, GG, F), jnp.float32),
        scratch_types=_s4_scratch() + [
            pltpu.VMEM((CSP,), jnp.int32),
            pltpu.VMEM((GG, F), jnp.float32),
            pltpu.SemaphoreType.DMA,
        ],
    )(_s4g_body)
    return kfn(m, dstp, counts, batch_pad)


def _s1a_body(pos_ref, w_ref, b_ref, u_ref, v_ref):
    wfull = w_ref[...]
    wh = wfull[0:3] + wfull[3:6]
    wp = wfull[3:6]
    p = pos_ref[...]
    u_ref[...] = jnp.dot(p, wh, preferred_element_type=jnp.float32) + b_ref[...]
    v_ref[...] = -jnp.dot(p, wp, preferred_element_type=jnp.float32)


def _run_s1a(pos_pad, w1, b1):
    grid = NPAD // BR1
    return pl.pallas_call(
        _s1a_body,
        grid=(grid,),
        in_specs=[
            pl.BlockSpec((BR1, 3), lambda i: (i, 0)),
            pl.BlockSpec((6, F), lambda i: (0, 0)),
            pl.BlockSpec((1, F), lambda i: (0, 0)),
        ],
        out_specs=[
            pl.BlockSpec((BR1, F), lambda i: (i, 0)),
            pl.BlockSpec((BR1, F), lambda i: (i, 0)),
        ],
        out_shape=[jax.ShapeDtypeStruct((NPAD, F), jnp.float32),
                   jax.ShapeDtypeStruct((NPAD, F), jnp.float32)],
    )(pos_pad, w1, b1)


def _s1b_body(h_ref, pos_ref, w_ref, b_ref, u_ref, v_ref):
    wfull = w_ref[...]
    wh = wfull[0:F]
    wp = wfull[F:F + 3]
    p = pos_ref[...]
    pv = jnp.dot(p, wp, preferred_element_type=jnp.float32)
    u_ref[...] = (jnp.dot(h_ref[...], wh, preferred_element_type=jnp.float32)
                  + pv + b_ref[...])
    v_ref[...] = -pv


def _run_s1b(h1, pos_pad, w1, b1):
    grid = NPAD // BR1
    return pl.pallas_call(
        _s1b_body,
        grid=(grid,),
        in_specs=[
            pl.BlockSpec((BR1, F), lambda i: (i, 0)),
            pl.BlockSpec((BR1, 3), lambda i: (i, 0)),
            pl.BlockSpec((F + 3, F), lambda i: (0, 0)),
            pl.BlockSpec((1, F), lambda i: (0, 0)),
        ],
        out_specs=[
            pl.BlockSpec((BR1, F), lambda i: (i, 0)),
            pl.BlockSpec((BR1, F), lambda i: (i, 0)),
        ],
        out_shape=[jax.ShapeDtypeStruct((NPAD, F), jnp.float32),
                   jax.ShapeDtypeStruct((NPAD, F), jnp.float32)],
    )(h1, pos_pad, w1, b1)


def _s3_body(z_ref, w_ref, b_ref, m_ref):
    m_ref[...] = (jnp.dot(z_ref[...], w_ref[...],
                          preferred_element_type=jnp.float32) + b_ref[...])


def _run_s3(z, w2bd, b2t):
    grid = EP4 // (BR3 // 4)
    return pl.pallas_call(
        _s3_body,
        grid=(grid,),
        in_specs=[
            pl.BlockSpec((BR3 // 4, 128), lambda i: (i, 0)),
            pl.BlockSpec((128, 128), lambda i: (0, 0)),
            pl.BlockSpec((1, 128), lambda i: (0, 0)),
        ],
        out_specs=pl.BlockSpec((BR3 // 4, 128), lambda i: (i, 0)),
        out_shape=jax.ShapeDtypeStruct((EP4, 128), jnp.float32),
    )(z, w2bd, b2t)


def _fin_body(gp_ref, wc_ref, bc_ref, out_ref):
    g = jnp.max(gp_ref[...], axis=0)
    out_ref[...] = (jnp.dot(g, wc_ref[...],
                            preferred_element_type=jnp.float32) + bc_ref[...])


def _run_fin(gpart, wc, bc):
    return pl.pallas_call(
        _fin_body,
        out_shape=jax.ShapeDtypeStruct((GG, wc.shape[1]), jnp.float32),
    )(gpart, wc, bc)


def kernel(pos, edge_index, batch, W1a, b1a, W2a, b2a, W1b, b1b, W2b, b2b,
           Wc, bc):
    src = edge_index[0]
    dst = edge_index[1]
    pos_pad = jnp.pad(pos, ((0, NPAD - NN), (0, 0)))
    batch_pad = jnp.pad(batch, (0, NPAD - NN))

    zf = jnp.zeros((F, F), jnp.float32)
    w2bd_a = jnp.block([[W2a, zf, zf, zf], [zf, W2a, zf, zf],
                        [zf, zf, W2a, zf], [zf, zf, zf, W2a]])
    w2bd_b = jnp.block([[W2b, zf, zf, zf], [zf, W2b, zf, zf],
                        [zf, zf, W2b, zf], [zf, zf, zf, W2b]])
    b2t_a = jnp.tile(b2a, 4).reshape(1, 128)
    b2t_b = jnp.tile(b2b, 4).reshape(1, 128)

    counts = _run_pa(dst)
    src_p, dst_p = _run_pc(src, dst, counts)

    u1, v1 = _run_s1a(pos_pad, W1a, b1a.reshape(1, F))
    z1 = _run_s2(src_p, dst_p, u1, v1)
    m1 = _run_s3(z1, w2bd_a, b2t_a)
    h1 = _run_s4h(m1, dst_p, counts)

    u2, v2 = _run_s1b(h1, pos_pad, W1b, b1b.reshape(1, F))
    z2 = _run_s2(src_p, dst_p, u2, v2)
    m2 = _run_s3(z2, w2bd_b, b2t_b)
    gpart = _run_s4g(m2, dst_p, counts, batch_pad)

    return _run_fin(gpart, Wc, bc.reshape(1, -1))


# speedup vs baseline: 5.1089x; 1.0120x over previous
import functools

import jax
import jax.numpy as jnp
from jax import lax
from jax.experimental import pallas as pl
from jax.experimental.pallas import tpu as pltpu
from jax.experimental.pallas import tpu_sc as plsc

NN = 100000
EE = 1600000
GG = 64
F = 32
NC, NS, LN = 2, 16, 16
NW = NC * NS
NB = 32
CSP = 3200
NPAD = NB * CSP
EPW = EE // NW
CH = 2000
NCH = EPW // CH
EPAD = 1671168
EPW2 = EPAD // NW
S2CB = 512
S2NF = EPW2 // S2CB
ESZ = EPAD
LSZ = EPW + NB * 64
BR1 = 6400
BR3 = 4096
EP4 = EPAD // 4
DW = 8

_SC_PARAMS = pltpu.CompilerParams(
    needs_layout_passes=False, use_tc_tiling_on_sc=False)


def _mesh():
    return plsc.VectorSubcoreMesh(
        core_axis_name="c", subcore_axis_name="s",
        num_cores=NC, num_subcores=NS)


def _wid():
    return lax.axis_index("s") * NC + lax.axis_index("c")


def _bkt(d):
    return ((d >> 7) * 5243) >> 17


def _lanes():
    return lax.iota(jnp.int32, LN)


def _prefix(cb, cs):
    lanes = _lanes()
    carry = jnp.int32(0)
    for k in range(NW * NB // LN):
        j = k * LN + lanes
        b = j >> 5
        wv = j & 31
        idx = wv * NB + b
        c = plsc.load_gather(cb, [idx])
        cp = (c + 63) & (-64)
        excl = plsc.cumsum(cp) - cp + carry
        plsc.store_scatter(cs, [idx], excl)
        carry = carry + jnp.sum(cp)
    return carry


def _pa_body(dst_hbm, counts_hbm, dbuf, hist, cbuf, sem):
    del sem
    w = _wid()
    lanes = _lanes()
    for b in range(NB):
        hist[pl.ds(b * LN, LN)] = jnp.zeros((LN,), jnp.int32)

    def chunk(c, car):
        pltpu.sync_copy(dst_hbm.at[pl.ds(pl.multiple_of(w * EPW + c * CH, 8), CH)], dbuf)

        def hb(v, car2):
            d = dbuf[pl.ds(v * LN, LN)]
            fi = _bkt(d) * LN + lanes
            plsc.store_scatter(hist, [fi], plsc.load_gather(hist, [fi]) + 1)
            return car2

        return lax.fori_loop(0, CH // LN, hb, car)

    lax.fori_loop(0, NCH, chunk, 0)
    for g in range(NB // LN):
        acc = jnp.zeros((LN,), jnp.int32)
        for l in range(LN):
            acc = acc + plsc.load_gather(hist, [(lanes + g * LN) * LN + l])
        cbuf[pl.ds(g * LN, LN)] = acc
    pltpu.sync_copy(cbuf, counts_hbm.at[pl.ds(pl.multiple_of(w * NB, 8), NB)])


def _run_pa(dst):
    kfn = functools.partial(
        pl.kernel, mesh=_mesh(), compiler_params=_SC_PARAMS,
        out_type=jax.ShapeDtypeStruct((NW * NB,), jnp.int32),
        scratch_types=[
            pltpu.VMEM((CH,), jnp.int32),
            pltpu.VMEM((NB * LN,), jnp.int32),
            pltpu.VMEM((NB,), jnp.int32),
            pltpu.SemaphoreType.DMA,
        ],
    )(_pa_body)
    return kfn(dst)


def _pc_body(src_hbm, dst_hbm, counts_hbm, srcp_hbm, dstp_hbm,
             dbuf, sbuf, cb, cs, fl, ll0, loc_s, loc_d, sem):
    w = _wid()
    lanes = _lanes()
    pltpu.sync_copy(counts_hbm, cb)
    _prefix(cb, cs)
    lcarry = jnp.int32(0)
    for g in range(NB // LN):
        c = cb[pl.ds(w * NB + g * LN, LN)]
        cp = (c + 63) & (-64)
        excl = plsc.cumsum(cp) - cp + lcarry
        fl[pl.ds(g * LN, LN)] = excl
        ll0[pl.ds(g * LN, LN)] = excl
        lcarry = lcarry + jnp.sum(cp)

    def chunk(c, car):
        base_e = pl.multiple_of(w * EPW + c * CH, 8)
        pltpu.sync_copy(dst_hbm.at[pl.ds(base_e, CH)], dbuf)
        pltpu.sync_copy(src_hbm.at[pl.ds(base_e, CH)], sbuf)

        def vec(v, car2):
            d = dbuf[pl.ds(v * LN, LN)]
            s = sbuf[pl.ds(v * LN, LN)]
            kd, pv = plsc.sort_key_val(d, lanes)
            sv = s.at[pv].get(mode="promise_in_bounds")
            b = _bkt(kd)
            bprev = b.at[jnp.maximum(lanes - 1, 0)].get(
                mode="promise_in_bounds")
            mnew = (b != bprev) | (lanes == 0)
            runstart = plsc.cummax(jnp.where(mnew, lanes, -1))
            rank = lanes - runstart
            pos = plsc.load_gather(fl, [b]) + rank
            plsc.store_scatter(loc_s, [pos], sv)
            plsc.store_scatter(loc_d, [pos], kd)
            bnext = b.at[jnp.minimum(lanes + 1, LN - 1)].get(
                mode="promise_in_bounds")
            mend = (b != bnext) | (lanes == LN - 1)
            plsc.store_scatter(fl, [b], pos + 1, mask=mend)
            return car2

        lax.fori_loop(0, CH // LN, vec, 0)
        return car

    lax.fori_loop(0, NCH, chunk, 0)

    for b in range(NB):
        g0 = (b // LN) * LN
        f = fl[pl.ds(g0, LN)][b % LN]
        l0 = ll0[pl.ds(g0, LN)][b % LN]
        cnt_b = f - l0
        cpb = (cnt_b + 63) & (-64)
        npad = cpb - cnt_b
        lastix = jnp.maximum(f - 1, 0)
        vs = plsc.load_gather(loc_s, [jnp.full((LN,), lastix, jnp.int32)])
        vd = plsc.load_gather(loc_d, [jnp.full((LN,), lastix, jnp.int32)])
        for t in range(4):
            tl = t * LN + lanes
            idxp = f + tl
            mpad = tl < npad
            plsc.store_scatter(loc_s, [idxp], vs, mask=mpad)
            plsc.store_scatter(loc_d, [idxp], vd, mask=mpad)
        gc0 = plsc.load_gather(
            cs, [jnp.full((LN,), w * NB + b, jnp.int32)])[0]
        n64 = cpb >> 6

        def is64(j, car, l0=l0, gc0=gc0):
            so = pl.multiple_of(l0 + j * 64, 8)
            do = pl.multiple_of(gc0 + j * 64, 8)
            pltpu.async_copy(loc_s.at[pl.ds(so, 64)],
                             srcp_hbm.at[pl.ds(do, 64)], sem)
            pltpu.async_copy(loc_d.at[pl.ds(so, 64)],
                             dstp_hbm.at[pl.ds(do, 64)], sem)

            @pl.when(j >= DW)
            def _drain(j=j, l0=l0, gc0=gc0):
                sod = pl.multiple_of(l0 + (j - DW) * 64, 8)
                dod = pl.multiple_of(gc0 + (j - DW) * 64, 8)
                pltpu.make_async_copy(
                    loc_s.at[pl.ds(sod, 64)],
                    srcp_hbm.at[pl.ds(dod, 64)], sem).wait()
                pltpu.make_async_copy(
                    loc_d.at[pl.ds(sod, 64)],
                    dstp_hbm.at[pl.ds(dod, 64)], sem).wait()
            return car

        lax.fori_loop(0, n64, is64, 0)

        def drain64(j, car, l0=l0, gc0=gc0):
            sod = pl.multiple_of(l0 + j * 64, 8)
            dod = pl.multiple_of(gc0 + j * 64, 8)
            pltpu.make_async_copy(
                loc_s.at[pl.ds(sod, 64)],
                srcp_hbm.at[pl.ds(dod, 64)], sem).wait()
            pltpu.make_async_copy(
                loc_d.at[pl.ds(sod, 64)],
                dstp_hbm.at[pl.ds(dod, 64)], sem).wait()
            return car

        lax.fori_loop(jnp.maximum(n64 - DW, 0), n64, drain64, 0)


def _run_pc(src, dst, counts):
    kfn = functools.partial(
        pl.kernel, mesh=_mesh(), compiler_params=_SC_PARAMS,
        out_type=[jax.ShapeDtypeStruct((ESZ,), jnp.int32),
                  jax.ShapeDtypeStruct((ESZ,), jnp.int32)],
        scratch_types=[
            pltpu.VMEM((CH,), jnp.int32),
            pltpu.VMEM((CH,), jnp.int32),
            pltpu.VMEM((NW * NB,), jnp.int32),
            pltpu.VMEM((NW * NB,), jnp.int32),
            pltpu.VMEM((NB,), jnp.int32),
            pltpu.VMEM((NB,), jnp.int32),
            pltpu.VMEM((LSZ,), jnp.int32),
            pltpu.VMEM((LSZ,), jnp.int32),
            pltpu.SemaphoreType.DMA,
        ],
    )(_pc_body)
    return kfn(src, dst, counts)


def _s2_body(srcp_hbm, dstp_hbm, u_hbm, v_hbm, z_hbm,
             sidx, didx, ubuf, vbuf, zbuf, isem, gsem_a, gsem_b):
    w = _wid()
    base = w * EPW2
    nc2 = S2NF // 2
    gsems = (gsem_a, gsem_b)

    def prefetch(eoff, p):
        for j in range(4):
            off_j = pl.multiple_of(eoff + j * 128, 8)
            pltpu.async_copy(srcp_hbm.at[pl.ds(off_j, 128)],
                             sidx.at[p * 4 + j], isem)
            pltpu.async_copy(dstp_hbm.at[pl.ds(off_j, 128)],
                             didx.at[p * 4 + j], isem)
        for j in range(4):
            off_j = pl.multiple_of(eoff + j * 128, 8)
            pltpu.make_async_copy(srcp_hbm.at[pl.ds(off_j, 128)],
                                  sidx.at[p * 4 + j], isem).wait()
            pltpu.make_async_copy(dstp_hbm.at[pl.ds(off_j, 128)],
                                  didx.at[p * 4 + j], isem).wait()
        for j in range(4):
            for k in range(8):
                iv = sidx[p * 4 + j, pl.ds(k * LN, LN)]
                sidx[p * 4 + j, pl.ds(k * LN, LN)] = jnp.minimum(
                    jnp.maximum(iv, 0), NPAD - 1)
                iv2 = didx[p * 4 + j, pl.ds(k * LN, LN)]
                didx[p * 4 + j, pl.ds(k * LN, LN)] = jnp.minimum(
                    jnp.maximum(iv2, 0), NPAD - 1)
        for j in range(4):
            pltpu.async_copy(u_hbm.at[sidx.at[p * 4 + j]],
                             ubuf.at[pl.ds(p * 512 + j * 128, 128)], gsems[p])
            pltpu.async_copy(v_hbm.at[didx.at[p * 4 + j]],
                             vbuf.at[pl.ds(p * 512 + j * 128, 128)], gsems[p])

    def consume(eoff, p):
        for j in range(4):
            pltpu.make_async_copy(
                u_hbm.at[sidx.at[p * 4 + j]],
                ubuf.at[pl.ds(p * 512 + j * 128, 128)], gsems[p]).wait()
            pltpu.make_async_copy(
                v_hbm.at[didx.at[p * 4 + j]],
                vbuf.at[pl.ds(p * 512 + j * 128, 128)], gsems[p]).wait()

        def cz(i2, car):
            for q in range(4):
                for h in range(2):
                    e = p * 512 + i2 * 4 + q
                    zbuf[i2, pl.ds(q * 32 + h * LN, LN)] = jnp.maximum(
                        ubuf[e, pl.ds(h * LN, LN)] + vbuf[e, pl.ds(h * LN, LN)],
                        0.0)
            return car

        lax.fori_loop(0, 128, cz, 0)
        pltpu.sync_copy(
            zbuf, z_hbm.at[pl.ds(pl.multiple_of(eoff >> 2, 8), 128)])

    prefetch(base, 0)

    def loop(c2, car):
        e0 = base + (2 * c2) * S2CB
        e1 = base + (2 * c2 + 1) * S2CB
        prefetch(e1, 1)
        consume(e0, 0)

        @pl.when(c2 < nc2 - 1)
        def _pf():
            prefetch(e1 + S2CB, 0)
        consume(e1, 1)
        return car

    lax.fori_loop(0, nc2, loop, 0)


def _run_s2(srcp, dstp, u, v):
    kfn = functools.partial(
        pl.kernel, mesh=_mesh(), compiler_params=_SC_PARAMS,
        out_type=jax.ShapeDtypeStruct((EP4, 128), jnp.float32),
        scratch_types=[
            pltpu.VMEM((8, 128), jnp.int32),
            pltpu.VMEM((8, 128), jnp.int32),
            pltpu.VMEM((1024, F), jnp.float32),
            pltpu.VMEM((1024, F), jnp.float32),
            pltpu.VMEM((128, 128), jnp.float32),
            pltpu.SemaphoreType.DMA,
            pltpu.SemaphoreType.DMA,
            pltpu.SemaphoreType.DMA,
        ],
    )(_s2_body)
    return kfn(srcp, dstp, u, v)


def _s4_common(m_hbm, dstp_hbm, counts_hbm, cb, cs, mbuf, dbuf4, tbl, sem):
    w = _wid()
    pltpu.sync_copy(counts_hbm, cb)
    total = _prefix(cb, cs)
    s = plsc.load_gather(cs, [jnp.full((LN,), w, jnp.int32)])[0]
    e_next = plsc.load_gather(
        cs, [jnp.full((LN,), jnp.minimum(w + 1, NB - 1), jnp.int32)])[0]
    e = jnp.where(w == NB - 1, total, e_next)
    nodebase = w * CSP

    def zt(i, car):
        for h in range(2):
            tbl[i, pl.ds(h * LN, LN)] = jnp.zeros((LN,), jnp.float32)
        return car

    lax.fori_loop(0, CSP, zt, 0)
    dbuf4[pl.ds(0, LN)] = jnp.full((LN,), nodebase, jnp.int32)
    for r in range(4):
        for kk in range(8):
            mbuf[r, pl.ds(kk * LN, LN)] = jnp.zeros((LN,), jnp.float32)

    def apply_grp(g, p):
        dv = dbuf4[pl.ds(p * 256 + g * LN, LN)]
        dv = jnp.minimum(jnp.maximum(dv - nodebase, 0), CSP - 1)
        for j in range(LN):
            r = dv[j]
            mrow = p * 64 + g * 4 + (j >> 2)
            for h in range(2):
                mcol = (j & 3) * 32 + h * LN
                tbl[r, pl.ds(h * LN, LN)] = jnp.maximum(
                    tbl[r, pl.ds(h * LN, LN)], mbuf[mrow, pl.ds(mcol, LN)])

    cnt = e - s
    nfull = cnt >> 8

    def fire(c, p):
        off = pl.multiple_of(s + c * 256, 8)
        moff = pl.multiple_of((s + c * 256) >> 2, 8)
        pltpu.async_copy(m_hbm.at[pl.ds(moff, 64)],
                         mbuf.at[pl.ds(p * 64, 64)], sem)
        pltpu.async_copy(dstp_hbm.at[pl.ds(off, 256)],
                         dbuf4.at[pl.ds(p * 256, 256)], sem)

    def drain(c, p):
        off = pl.multiple_of(s + c * 256, 8)
        moff = pl.multiple_of((s + c * 256) >> 2, 8)
        pltpu.make_async_copy(m_hbm.at[pl.ds(moff, 64)],
                              mbuf.at[pl.ds(p * 64, 64)], sem).wait()
        pltpu.make_async_copy(dstp_hbm.at[pl.ds(off, 256)],
                              dbuf4.at[pl.ds(p * 256, 256)], sem).wait()

    @pl.when(nfull > 0)
    def _p0():
        fire(0, 0)

    def chunk(c, car):
        p = c & 1
        drain(c, p)

        @pl.when(c + 1 < nfull)
        def _pf(c=c, p=p):
            fire(c + 1, 1 - p)

        def grp(g, car2):
            apply_grp(g, p)
            return car2

        lax.fori_loop(0, 256 // LN, grp, 0)
        return car

    lax.fori_loop(0, nfull, chunk, 0)
    t0 = s + (nfull << 8)
    ng8 = (e - t0) >> 3

    def g8(j, car):
        off = pl.multiple_of(t0 + j * 8, 8)
        pltpu.sync_copy(m_hbm.at[pl.ds((t0 + j * 8) >> 2, 2)],
                        mbuf.at[pl.ds(0, 2)])
        pltpu.sync_copy(dstp_hbm.at[pl.ds(off, 8)], dbuf4.at[pl.ds(0, 8)])
        apply_grp(0, 0)
        return car

    lax.fori_loop(0, ng8, g8, 0)
    return w, nodebase


def _s4h_body(m_hbm, dstp_hbm, counts_hbm, h_hbm,
              cb, cs, mbuf, dbuf4, tbl, sem):
    w, nodebase = _s4_common(
        m_hbm, dstp_hbm, counts_hbm, cb, cs, mbuf, dbuf4, tbl, sem)
    pltpu.sync_copy(tbl, h_hbm.at[pl.ds(pl.multiple_of(nodebase, 8), CSP)])


def _s4g_body(m_hbm, dstp_hbm, counts_hbm, batch_hbm, gpart_hbm,
              cb, cs, mbuf, dbuf4, tbl, bbuf, gtbl, sem):
    w, nodebase = _s4_common(
        m_hbm, dstp_hbm, counts_hbm, cb, cs, mbuf, dbuf4, tbl, sem)
    pltpu.sync_copy(batch_hbm.at[pl.ds(pl.multiple_of(nodebase, 8), CSP)], bbuf)
    for r in range(GG):
        for h in range(2):
            gtbl[r, pl.ds(h * LN, LN)] = jnp.zeros((LN,), jnp.float32)

    def pool(rg, car):
        bv = bbuf[pl.ds(rg * LN, LN)]
        bv = jnp.minimum(jnp.maximum(bv, 0), GG - 1)
        for j in range(LN):
            gi = bv[j]
            nr = rg * LN + j
            for h in range(2):
                gtbl[gi, pl.ds(h * LN, LN)] = jnp.maximum(
                    gtbl[gi, pl.ds(h * LN, LN)], tbl[nr, pl.ds(h * LN, LN)])
        return car

    lax.fori_loop(0, CSP // LN, pool, 0)
    pltpu.sync_copy(gtbl, gpart_hbm.at[w])


def _s4_scratch():
    return [
        pltpu.VMEM((NW * NB,), jnp.int32),
        pltpu.VMEM((NW * NB,), jnp.int32),
        pltpu.VMEM((128, 128), jnp.float32),
        pltpu.VMEM((512,), jnp.int32),
        pltpu.VMEM((CSP, F), jnp.float32),
    ]


def _run_s4h(m, dstp, counts):
    kfn = functools.partial(
        pl.kernel, mesh=_mesh(), compiler_params=_SC_PARAMS,
        out_type=jax.ShapeDtypeStruct((NPAD, F), jnp.float32),
        scratch_types=_s4_scratch() + [pltpu.SemaphoreType.DMA],
    )(_s4h_body)
    return kfn(m, dstp, counts)


def _run_s4g(m, dstp, counts, batch_pad):
    kfn = functools.partial(
        pl.kernel, mesh=_mesh(), compiler_params=_SC_PARAMS,
        out_type=jax.ShapeDtypeStruct((NW, GG, F), jnp.float32),
        scratch_types=_s4_scratch() + [
            pltpu.VMEM((CSP,), jnp.int32),
            pltpu.VMEM((GG, F), jnp.float32),
            pltpu.SemaphoreType.DMA,
        ],
    )(_s4g_body)
    return kfn(m, dstp, counts, batch_pad)


def _s1a_body(pos_ref, w_ref, b_ref, u_ref, v_ref):
    wfull = w_ref[...]
    wh = wfull[0:3] + wfull[3:6]
    wp = wfull[3:6]
    p = pos_ref[...]
    u_ref[...] = jnp.dot(p, wh, preferred_element_type=jnp.float32) + b_ref[...]
    v_ref[...] = -jnp.dot(p, wp, preferred_element_type=jnp.float32)


def _run_s1a(pos_pad, w1, b1):
    grid = NPAD // BR1
    return pl.pallas_call(
        _s1a_body,
        grid=(grid,),
        in_specs=[
            pl.BlockSpec((BR1, 3), lambda i: (i, 0)),
            pl.BlockSpec((6, F), lambda i: (0, 0)),
            pl.BlockSpec((1, F), lambda i: (0, 0)),
        ],
        out_specs=[
            pl.BlockSpec((BR1, F), lambda i: (i, 0)),
            pl.BlockSpec((BR1, F), lambda i: (i, 0)),
        ],
        out_shape=[jax.ShapeDtypeStruct((NPAD, F), jnp.float32),
                   jax.ShapeDtypeStruct((NPAD, F), jnp.float32)],
    )(pos_pad, w1, b1)


def _s1b_body(h_ref, pos_ref, w_ref, b_ref, u_ref, v_ref):
    wfull = w_ref[...]
    wh = wfull[0:F]
    wp = wfull[F:F + 3]
    p = pos_ref[...]
    pv = jnp.dot(p, wp, preferred_element_type=jnp.float32)
    u_ref[...] = (jnp.dot(h_ref[...], wh, preferred_element_type=jnp.float32)
                  + pv + b_ref[...])
    v_ref[...] = -pv


def _run_s1b(h1, pos_pad, w1, b1):
    grid = NPAD // BR1
    return pl.pallas_call(
        _s1b_body,
        grid=(grid,),
        in_specs=[
            pl.BlockSpec((BR1, F), lambda i: (i, 0)),
            pl.BlockSpec((BR1, 3), lambda i: (i, 0)),
            pl.BlockSpec((F + 3, F), lambda i: (0, 0)),
            pl.BlockSpec((1, F), lambda i: (0, 0)),
        ],
        out_specs=[
            pl.BlockSpec((BR1, F), lambda i: (i, 0)),
            pl.BlockSpec((BR1, F), lambda i: (i, 0)),
        ],
        out_shape=[jax.ShapeDtypeStruct((NPAD, F), jnp.float32),
                   jax.ShapeDtypeStruct((NPAD, F), jnp.float32)],
    )(h1, pos_pad, w1, b1)


def _s3_body(z_ref, w_ref, b_ref, m_ref):
    m_ref[...] = (jnp.dot(z_ref[...], w_ref[...],
                          preferred_element_type=jnp.float32) + b_ref[...])


def _run_s3(z, w2bd, b2t):
    grid = EP4 // (BR3 // 4)
    return pl.pallas_call(
        _s3_body,
        grid=(grid,),
        in_specs=[
            pl.BlockSpec((BR3 // 4, 128), lambda i: (i, 0)),
            pl.BlockSpec((128, 128), lambda i: (0, 0)),
            pl.BlockSpec((1, 128), lambda i: (0, 0)),
        ],
        out_specs=pl.BlockSpec((BR3 // 4, 128), lambda i: (i, 0)),
        out_shape=jax.ShapeDtypeStruct((EP4, 128), jnp.float32),
    )(z, w2bd, b2t)


def _fin_body(gp_ref, wc_ref, bc_ref, out_ref):
    g = jnp.max(gp_ref[...], axis=0)
    out_ref[...] = (jnp.dot(g, wc_ref[...],
                            preferred_element_type=jnp.float32) + bc_ref[...])


def _run_fin(gpart, wc, bc):
    return pl.pallas_call(
        _fin_body,
        out_shape=jax.ShapeDtypeStruct((GG, wc.shape[1]), jnp.float32),
    )(gpart, wc, bc)


def kernel(pos, edge_index, batch, W1a, b1a, W2a, b2a, W1b, b1b, W2b, b2b,
           Wc, bc):
    src = edge_index[0]
    dst = edge_index[1]
    pos_pad = jnp.pad(pos, ((0, NPAD - NN), (0, 0)))
    batch_pad = jnp.pad(batch, (0, NPAD - NN))

    zf = jnp.zeros((F, F), jnp.float32)
    w2bd_a = jnp.block([[W2a, zf, zf, zf], [zf, W2a, zf, zf],
                        [zf, zf, W2a, zf], [zf, zf, zf, W2a]])
    w2bd_b = jnp.block([[W2b, zf, zf, zf], [zf, W2b, zf, zf],
                        [zf, zf, W2b, zf], [zf, zf, zf, W2b]])
    b2t_a = jnp.tile(b2a, 4).reshape(1, 128)
    b2t_b = jnp.tile(b2b, 4).reshape(1, 128)

    counts = _run_pa(dst)
    src_p, dst_p = _run_pc(src, dst, counts)

    u1, v1 = _run_s1a(pos_pad, W1a, b1a.reshape(1, F))
    z1 = _run_s2(src_p, dst_p, u1, v1)
    m1 = _run_s3(z1, w2bd_a, b2t_a)
    h1 = _run_s4h(m1, dst_p, counts)

    u2, v2 = _run_s1b(h1, pos_pad, W1b, b1b.reshape(1, F))
    z2 = _run_s2(src_p, dst_p, u2, v2)
    m2 = _run_s3(z2, w2bd_b, b2t_b)
    gpart = _run_s4g(m2, dst_p, counts, batch_pad)

    return _run_fin(gpart, Wc, bc.reshape(1, -1))
